# baseline (speedup 1.0000x reference)
import functools

import jax
import jax.numpy as jnp
from jax import lax
from jax.experimental import pallas as pl
from jax.experimental.pallas import tpu as pltpu
from jax.experimental.pallas import tpu_sc as plsc

EPS = 1e-5
K = 128


def _cwl(cw):
    return -(-(cw - 2) // 4) * 4 + 2


def _sc_dims():
    try:
        info = plsc.get_sparse_core_info()
        return info.num_cores, info.num_subcores
    except ValueError:
        return 2, 16


@functools.lru_cache(maxsize=None)
def _seg_sum_rows(T, S, CW, D, col_split):
    NC, NS = _sc_dims()
    assert S % 8 == 0
    DW = D // 2 if col_split else D
    base_rows = (S // NS) // 8 * 8
    tail = S - NS * base_rows
    mesh = plsc.VectorSubcoreMesh(core_axis_name="c", subcore_axis_name="s",
                                  num_cores=NC, num_subcores=NS)
    gshape = (NC, NS, CW + 3, K) if col_split else (NC * NS, CW + 3, K)
    sshape = (NS, CW, K) if col_split else (NC * NS, CW, K)

    @functools.partial(
        pl.kernel,
        out_type=jax.ShapeDtypeStruct((NC, S, DW), jnp.float32),
        mesh=mesh,
        scratch_types=[
            pltpu.VMEM((_cwl(CW) + 1, K), jnp.int32),
            pltpu.VMEM((_cwl(CW), K), jnp.int32),
            pltpu.VMEM((K, DW), jnp.float32),
            pltpu.VMEM((K, DW), jnp.float32),
            pltpu.VMEM((K, DW), jnp.float32),
            pltpu.VMEM((K, DW), jnp.float32),
            pltpu.VMEM((K, DW), jnp.float32),
            pltpu.VMEM_SHARED((S + K, DW), jnp.float32),
            pltpu.SemaphoreType.DMA,
            pltpu.SemaphoreType.DMA,
        ],
        compiler_params=pltpu.CompilerParams(use_tc_tiling_on_sc=False),
    )
    def k(tab, gidx, sidx, zeros, out, gbuf, sbuf, rb0, rb1, rb2, rb3, zbuf,
          acc, semg, sems):
        bufs = (rb0, rb1, rb2, rb3)
        c = lax.axis_index("c")
        s = lax.axis_index("s")

        pltpu.sync_copy(zeros, zbuf)
        row0 = s * base_rows
        for off in range(0, base_rows, K):
            sz = min(K, base_rows - off)
            pltpu.sync_copy(zbuf.at[pl.ds(0, sz)], acc.at[pl.ds(row0 + off, sz)])
        if tail:
            @pl.when(s == NS - 1)
            def _():
                pltpu.sync_copy(zbuf.at[pl.ds(0, tail)],
                                acc.at[pl.ds(NS * base_rows, tail)])
        plsc.subcore_barrier()

        if col_split:
            pltpu.sync_copy(gidx.at[c, s], gbuf)
            pltpu.sync_copy(sidx.at[s], sbuf)
        else:
            w = s * NC + c
            pltpu.sync_copy(gidx.at[w], gbuf)
            pltpu.sync_copy(sidx.at[w], sbuf)

        CWL = _cwl(CW)

        def start_g(j, buf):
            pltpu.async_copy(tab.at[gbuf.at[j]], buf, semg)

        def wait_g(j, buf):
            pltpu.make_async_copy(tab.at[gbuf.at[j]], buf, semg).wait()

        def start_s(j, buf):
            pltpu.async_copy(buf, acc.at[sbuf.at[j]], sems, add=True)

        def wait_s(j, buf):
            pltpu.make_async_copy(buf, acc.at[sbuf.at[j]], sems).wait()

        def step(j, u):
            wait_s(j - 2, bufs[(u + 2) % 4])
            start_g(j + 1, bufs[(u + 1) % 4])
            wait_g(j, bufs[u])
            start_s(j, bufs[u])

        start_g(0, rb0)
        start_g(1, rb1)
        start_g(2, rb2)
        wait_g(0, rb0)
        start_s(0, rb0)
        wait_g(1, rb1)
        start_s(1, rb1)

        def body(i, carry):
            j = 4 * i + 2
            for u in range(4):
                step(j + u, (u + 2) % 4)
            return carry

        lax.fori_loop(0, (CWL - 2) // 4, body, 0)
        wait_s(CWL - 2, bufs[(CWL - 2) % 4])
        wait_s(CWL - 1, bufs[(CWL - 1) % 4])
        wait_g(CWL, bufs[CWL % 4])
        plsc.subcore_barrier()

        def wout(r0, sz):
            pltpu.sync_copy(acc.at[pl.ds(r0, sz)], rb0.at[pl.ds(0, sz)])
            pltpu.sync_copy(rb0.at[pl.ds(0, sz)], out.at[c, pl.ds(r0, sz)])

        for off in range(0, base_rows, K):
            wout(row0 + off, min(K, base_rows - off))
        if tail:
            @pl.when(s == NS - 1)
            def _():
                wout(NS * base_rows, tail)

    return k


@functools.lru_cache(maxsize=None)
def _degrees(NN, EE, CW):
    NC, NS = _sc_dims()
    NW = NC * NS
    DD = 16
    assert CW % 2 == 1

    def plan(S):
        base_rows = (S // NS) // 8 * 8
        return base_rows, S - NS * base_rows

    mesh = plsc.VectorSubcoreMesh(core_axis_name="c", subcore_axis_name="s",
                                  num_cores=NC, num_subcores=NS)

    @functools.partial(
        pl.kernel,
        out_type=(jax.ShapeDtypeStruct((NC, NN, DD), jnp.float32),
                  jax.ShapeDtypeStruct((NC, EE, DD), jnp.float32)),
        mesh=mesh,
        scratch_types=[
            pltpu.VMEM((CW, K), jnp.int32),
            pltpu.VMEM((CW, K), jnp.int32),
            pltpu.VMEM((K, DD), jnp.float32),
            pltpu.VMEM((K, DD), jnp.float32),
            pltpu.VMEM((K, DD), jnp.float32),
            pltpu.VMEM((K, DD), jnp.float32),
            pltpu.VMEM_SHARED((NN + K, DD), jnp.float32),
            pltpu.VMEM_SHARED((EE + K, DD), jnp.float32),
            pltpu.SemaphoreType.DMA,
        ],
        compiler_params=pltpu.CompilerParams(use_tc_tiling_on_sc=False),
    )
    def k(wtab, hidx, sidx, ones, zeros, dn, de, hbuf, sbuf, rb0, rb1, onesb,
          zbuf, accn, acce, sem):
        c = lax.axis_index("c")
        s = lax.axis_index("s")
        w = s * NC + c

        pltpu.sync_copy(zeros, zbuf)
        pltpu.sync_copy(ones, onesb)
        for acc, S in ((accn, NN), (acce, EE)):
            base_rows, tail = plan(S)
            row0 = s * base_rows
            for off in range(0, base_rows, K):
                sz = min(K, base_rows - off)
                pltpu.sync_copy(zbuf.at[pl.ds(0, sz)], acc.at[pl.ds(row0 + off, sz)])
            if tail:
                @pl.when(s == NS - 1)
                def _():
                    pltpu.sync_copy(zbuf.at[pl.ds(0, tail)],
                                    acc.at[pl.ds(NS * base_rows, tail)])
        plsc.subcore_barrier()

        pltpu.sync_copy(hidx.at[w], hbuf)
        pltpu.sync_copy(sidx.at[w], sbuf)

        def start_g(j, buf):
            pltpu.async_copy(wtab.at[hbuf.at[j]], buf, sem)

        def wait_g(j, buf):
            pltpu.make_async_copy(wtab.at[hbuf.at[j]], buf, sem).wait()

        def scat(j, buf):
            pltpu.sync_copy(buf, accn.at[sbuf.at[j]], add=True)
            pltpu.sync_copy(onesb, acce.at[hbuf.at[j]], add=True)

        start_g(0, rb0)

        def body(i, carry):
            j = 2 * i
            wait_g(j, rb0)
            start_g(j + 1, rb1)
            scat(j, rb0)
            wait_g(j + 1, rb1)
            start_g(j + 2, rb0)
            scat(j + 1, rb1)
            return carry

        lax.fori_loop(0, (CW - 1) // 2, body, 0)
        wait_g(CW - 1, rb0)
        scat(CW - 1, rb0)
        plsc.subcore_barrier()

        for acc, S, out in ((accn, NN, dn), (acce, EE, de)):
            base_rows, tail = plan(S)
            row0 = s * base_rows

            def wout(r0, sz, acc=acc, out=out):
                pltpu.sync_copy(acc.at[pl.ds(r0, sz)], rb0.at[pl.ds(0, sz)])
                pltpu.sync_copy(rb0.at[pl.ds(0, sz)], out.at[c, pl.ds(r0, sz)])

            for off in range(0, base_rows, K):
                wout(row0 + off, min(K, base_rows - off))
            if tail:
                @pl.when(s == NS - 1)
                def _():
                    wout(NS * base_rows, tail)

    return k


def _safe_inv(d):
    return jnp.where(d > 0, 1.0 / jnp.where(d > 0, d, 1.0), 0.0)


def _matmul_t(x, W):
    n, f = x.shape
    blk = 1000
    assert n % blk == 0

    def body(x_ref, w_ref, o_ref):
        o_ref[...] = lax.dot_general(
            x_ref[...], w_ref[...], (((1,), (1,)), ((), ())),
            preferred_element_type=jnp.float32,
            precision=lax.Precision.HIGHEST)

    return pl.pallas_call(
        body,
        grid=(n // blk,),
        in_specs=[pl.BlockSpec((blk, f), lambda i: (i, 0)),
                  pl.BlockSpec((f, f), lambda i: (0, 0))],
        out_specs=pl.BlockSpec((blk, f), lambda i: (i, 0)),
        out_shape=jax.ShapeDtypeStruct((n, f), jnp.float32),
    )(x, W)


def _combine_scale(halves, deg_parts):
    _, s, dw = halves.shape
    d = 2 * dw
    dd = deg_parts.shape[2]
    blk = 1000
    assert s % blk == 0

    def body(p_ref, dg_ref, o_ref):
        deg = dg_ref[0, :, 0] + dg_ref[1, :, 0]
        full = jnp.concatenate([p_ref[0], p_ref[1]], axis=1)
        o_ref[...] = full * _safe_inv(deg)[:, None]

    return pl.pallas_call(
        body,
        grid=(s // blk,),
        in_specs=[pl.BlockSpec((2, blk, dw), lambda i: (0, i, 0)),
                  pl.BlockSpec((2, blk, dd), lambda i: (0, i, 0))],
        out_specs=pl.BlockSpec((blk, d), lambda i: (i, 0)),
        out_shape=jax.ShapeDtypeStruct((s, d), jnp.float32),
    )(halves, deg_parts)


def _combine_bias_tanh_stats(halves, deg_parts, b2):
    _, n, dw = halves.shape
    d = 2 * dw
    dd = deg_parts.shape[2]
    blk = 1000
    assert n % blk == 0

    def body(p_ref, dg_ref, b_ref, t_ref, s_ref):
        i = pl.program_id(0)
        deg = dg_ref[0, :, 0] + dg_ref[1, :, 0]
        full = jnp.concatenate([p_ref[0], p_ref[1]], axis=1)
        z = full * _safe_inv(deg)[:, None] + b_ref[...]
        t = jnp.tanh(z)
        t_ref[...] = t
        st = jnp.concatenate(
            [jnp.sum(t, 0, keepdims=True),
             jnp.sum(t * t, 0, keepdims=True),
             jnp.zeros((6, d), jnp.float32)], axis=0)

        @pl.when(i == 0)
        def _():
            s_ref[...] = jnp.zeros_like(s_ref)

        s_ref[...] = s_ref[...] + st

    return pl.pallas_call(
        body,
        grid=(n // blk,),
        in_specs=[pl.BlockSpec((2, blk, dw), lambda i: (0, i, 0)),
                  pl.BlockSpec((2, blk, dd), lambda i: (0, i, 0)),
                  pl.BlockSpec((1, d), lambda i: (0, 0))],
        out_specs=[pl.BlockSpec((blk, d), lambda i: (i, 0)),
                   pl.BlockSpec((8, d), lambda i: (0, 0))],
        out_shape=[jax.ShapeDtypeStruct((n, d), jnp.float32),
                   jax.ShapeDtypeStruct((8, d), jnp.float32)],
    )(halves, deg_parts, b2)


def _batchnorm_apply(t, sums, g2, beta2):
    n, d = t.shape
    blk = 1000
    assert n % blk == 0
    inv_n = 1.0 / n

    def body(t_ref, s_ref, g_ref, be_ref, o_ref):
        m = s_ref[0, :] * inv_n
        v = s_ref[1, :] * inv_n - m * m
        scale = lax.rsqrt(v + EPS) * g_ref[0, :]
        o_ref[...] = (t_ref[...] - m[None, :]) * scale[None, :] + be_ref[...]

    return pl.pallas_call(
        body,
        grid=(n // blk,),
        in_specs=[pl.BlockSpec((blk, d), lambda i: (i, 0)),
                  pl.BlockSpec((8, d), lambda i: (0, 0)),
                  pl.BlockSpec((1, d), lambda i: (0, 0)),
                  pl.BlockSpec((1, d), lambda i: (0, 0))],
        out_specs=pl.BlockSpec((blk, d), lambda i: (i, 0)),
        out_shape=jax.ShapeDtypeStruct((n, d), jnp.float32),
    )(t, sums, g2, beta2)


def kernel(x, edge_index, weight, W0, b0, g0, beta0, W1, b1, g1, beta1):
    n, f = x.shape
    nnz = edge_index.shape[1]
    eh = weight.shape[0]
    fw = f // 2
    NC, NS = _sc_dims()
    NW = NC * NS

    def pad_to(idx, nchunks, val):
        npad = nchunks * K - nnz
        return jnp.concatenate([idx, jnp.full((npad,), val, jnp.int32)])

    cw16 = -(-nnz // (NS * K))
    cw16 += 1 - cw16 % 2
    g_ch = _cwl(cw16) + 1
    s_ch = _cwl(cw16)

    def ext(a, nch, val):
        pad = jnp.full((NS, nch - cw16, K), val, jnp.int32)
        return jnp.concatenate([a, pad], axis=1)

    src16_g = ext(pad_to(edge_index[0], NS * cw16, 0).reshape(NS, cw16, K), g_ch, 0)
    src16_s = ext(pad_to(edge_index[0], NS * cw16, n).reshape(NS, cw16, K), s_ch, n)
    he16_g = ext(pad_to(edge_index[1], NS * cw16, 0).reshape(NS, cw16, K), g_ch, 0)
    he16_s = ext(pad_to(edge_index[1], NS * cw16, eh).reshape(NS, cw16, K), s_ch, eh)
    src_cg = jnp.stack([2 * src16_g, 2 * src16_g + 1])
    he_cg = jnp.stack([2 * he16_g, 2 * he16_g + 1])

    zeros_h = jnp.zeros((K, fw), jnp.float32)

    cw32 = -(-nnz // (NW * K))
    cw32 += 1 - cw32 % 2
    he32 = pad_to(edge_index[1], NW * cw32, eh).reshape(NW, cw32, K)
    src32 = pad_to(edge_index[0], NW * cw32, n).reshape(NW, cw32, K)
    tab_w16 = jnp.zeros((eh + K, 16), jnp.float32).at[:eh, 0].set(weight)
    ones16 = jnp.ones((K, 16), jnp.float32)
    zeros16 = jnp.zeros((K, 16), jnp.float32)

    seg = _seg_sum_rows(n, eh, cw16, f, True)

    dn_parts, de_parts = _degrees(n, eh, cw32)(
        tab_w16, he32, src32, ones16, zeros16)

    def layer(h, W, b, g, beta):
        xl = _matmul_t(h, W)
        pe = seg(xl.reshape(2 * n, fw), src_cg, he16_s, zeros_h)
        out_e = _combine_scale(pe, de_parts)
        pn = seg(out_e.reshape(2 * eh, fw), he_cg, src16_s, zeros_h)
        t, sums = _combine_bias_tanh_stats(pn, dn_parts, b.reshape(1, f))
        return _batchnorm_apply(t, sums, g.reshape(1, f), beta.reshape(1, f))

    h1 = layer(x, W0, b0, g0, beta0)
    h2 = layer(h1, W1, b1, g1, beta1)
    return jnp.stack([h1, h2])


# speedup vs baseline: 1.0580x; 1.0580x over previous
import functools

import jax
import jax.numpy as jnp
from jax import lax
from jax.experimental import pallas as pl
from jax.experimental.pallas import tpu as pltpu
from jax.experimental.pallas import tpu_sc as plsc

EPS = 1e-5
K = 128


def _cwl(cw):
    return -(-(cw - 2) // 4) * 4 + 2


def _sc_dims():
    try:
        info = plsc.get_sparse_core_info()
        return info.num_cores, info.num_subcores
    except ValueError:
        return 2, 16


@functools.lru_cache(maxsize=None)
def _seg_sum_rows(T, S, CW, D, col_split):
    NC, NS = _sc_dims()
    assert S % 8 == 0
    DW = D // 2 if col_split else D
    base_rows = (S // NS) // 8 * 8
    tail = S - NS * base_rows
    mesh = plsc.VectorSubcoreMesh(core_axis_name="c", subcore_axis_name="s",
                                  num_cores=NC, num_subcores=NS)
    gshape = (NC, NS, CW + 3, K) if col_split else (NC * NS, CW + 3, K)
    sshape = (NS, CW, K) if col_split else (NC * NS, CW, K)

    @functools.partial(
        pl.kernel,
        out_type=jax.ShapeDtypeStruct((NC, S, DW), jnp.float32),
        mesh=mesh,
        scratch_types=[
            pltpu.VMEM((_cwl(CW) + 1, K), jnp.int32),
            pltpu.VMEM((_cwl(CW), K), jnp.int32),
            pltpu.VMEM((K, DW), jnp.float32),
            pltpu.VMEM((K, DW), jnp.float32),
            pltpu.VMEM((K, DW), jnp.float32),
            pltpu.VMEM_SHARED((S + K, DW), jnp.float32),
            pltpu.SemaphoreType.DMA,
            pltpu.SemaphoreType.DMA,
        ],
        compiler_params=pltpu.CompilerParams(use_tc_tiling_on_sc=False),
    )
    def k(tab, gidx, sidx, zeros, out, gbuf, sbuf, rb0, rb1, zbuf, acc, semg,
          sems):
        c = lax.axis_index("c")
        s = lax.axis_index("s")

        pltpu.sync_copy(zeros, zbuf)
        row0 = s * base_rows
        for off in range(0, base_rows, K):
            sz = min(K, base_rows - off)
            pltpu.sync_copy(zbuf.at[pl.ds(0, sz)], acc.at[pl.ds(row0 + off, sz)])
        if tail:
            @pl.when(s == NS - 1)
            def _():
                pltpu.sync_copy(zbuf.at[pl.ds(0, tail)],
                                acc.at[pl.ds(NS * base_rows, tail)])
        plsc.subcore_barrier()

        if col_split:
            pltpu.sync_copy(gidx.at[c, s], gbuf)
            pltpu.sync_copy(sidx.at[s], sbuf)
        else:
            w = s * NC + c
            pltpu.sync_copy(gidx.at[w], gbuf)
            pltpu.sync_copy(sidx.at[w], sbuf)

        def start_g(j, buf):
            pltpu.async_copy(tab.at[gbuf.at[j]], buf, semg)

        def wait_g(j, buf):
            pltpu.make_async_copy(tab.at[gbuf.at[j]], buf, semg).wait()

        def start_s(j, buf):
            pltpu.async_copy(buf, acc.at[sbuf.at[j]], sems, add=True)

        def wait_s(j, buf):
            pltpu.make_async_copy(buf, acc.at[sbuf.at[j]], sems).wait()

        start_g(0, rb0)
        wait_g(0, rb0)
        start_s(0, rb0)
        start_g(1, rb1)

        def body(i, carry):
            j = 2 * i + 1
            wait_g(j, rb1)
            start_s(j, rb1)
            wait_s(j - 1, rb0)
            start_g(j + 1, rb0)
            wait_g(j + 1, rb0)
            start_s(j + 1, rb0)
            wait_s(j, rb1)
            start_g(j + 2, rb1)
            return carry

        lax.fori_loop(0, (CW - 1) // 2, body, 0)
        wait_s(CW - 1, rb0)
        wait_g(CW, rb1)
        plsc.subcore_barrier()

        def wout(r0, sz):
            pltpu.sync_copy(acc.at[pl.ds(r0, sz)], rb0.at[pl.ds(0, sz)])
            pltpu.sync_copy(rb0.at[pl.ds(0, sz)], out.at[c, pl.ds(r0, sz)])

        for off in range(0, base_rows, K):
            wout(row0 + off, min(K, base_rows - off))
        if tail:
            @pl.when(s == NS - 1)
            def _():
                wout(NS * base_rows, tail)

    return k


@functools.lru_cache(maxsize=None)
def _degrees(NN, EE, CW):
    NC, NS = _sc_dims()
    NW = NC * NS
    DD = 16
    assert CW % 2 == 1

    def plan(S):
        base_rows = (S // NS) // 8 * 8
        return base_rows, S - NS * base_rows

    mesh = plsc.VectorSubcoreMesh(core_axis_name="c", subcore_axis_name="s",
                                  num_cores=NC, num_subcores=NS)

    @functools.partial(
        pl.kernel,
        out_type=(jax.ShapeDtypeStruct((NC, NN, DD), jnp.float32),
                  jax.ShapeDtypeStruct((NC, EE, DD), jnp.float32)),
        mesh=mesh,
        scratch_types=[
            pltpu.VMEM((CW, K), jnp.int32),
            pltpu.VMEM((CW, K), jnp.int32),
            pltpu.VMEM((K, DD), jnp.float32),
            pltpu.VMEM((K, DD), jnp.float32),
            pltpu.VMEM((K, DD), jnp.float32),
            pltpu.VMEM((K, DD), jnp.float32),
            pltpu.VMEM_SHARED((NN + K, DD), jnp.float32),
            pltpu.VMEM_SHARED((EE + K, DD), jnp.float32),
            pltpu.SemaphoreType.DMA,
        ],
        compiler_params=pltpu.CompilerParams(use_tc_tiling_on_sc=False),
    )
    def k(wtab, hidx, sidx, ones, zeros, dn, de, hbuf, sbuf, rb0, rb1, onesb,
          zbuf, accn, acce, sem):
        c = lax.axis_index("c")
        s = lax.axis_index("s")
        w = s * NC + c

        pltpu.sync_copy(zeros, zbuf)
        pltpu.sync_copy(ones, onesb)
        for acc, S in ((accn, NN), (acce, EE)):
            base_rows, tail = plan(S)
            row0 = s * base_rows
            for off in range(0, base_rows, K):
                sz = min(K, base_rows - off)
                pltpu.sync_copy(zbuf.at[pl.ds(0, sz)], acc.at[pl.ds(row0 + off, sz)])
            if tail:
                @pl.when(s == NS - 1)
                def _():
                    pltpu.sync_copy(zbuf.at[pl.ds(0, tail)],
                                    acc.at[pl.ds(NS * base_rows, tail)])
        plsc.subcore_barrier()

        pltpu.sync_copy(hidx.at[w], hbuf)
        pltpu.sync_copy(sidx.at[w], sbuf)

        def start_g(j, buf):
            pltpu.async_copy(wtab.at[hbuf.at[j]], buf, sem)

        def wait_g(j, buf):
            pltpu.make_async_copy(wtab.at[hbuf.at[j]], buf, sem).wait()

        def scat(j, buf):
            pltpu.sync_copy(buf, accn.at[sbuf.at[j]], add=True)
            pltpu.sync_copy(onesb, acce.at[hbuf.at[j]], add=True)

        start_g(0, rb0)

        def body(i, carry):
            j = 2 * i
            wait_g(j, rb0)
            start_g(j + 1, rb1)
            scat(j, rb0)
            wait_g(j + 1, rb1)
            start_g(j + 2, rb0)
            scat(j + 1, rb1)
            return carry

        lax.fori_loop(0, (CW - 1) // 2, body, 0)
        wait_g(CW - 1, rb0)
        scat(CW - 1, rb0)
        plsc.subcore_barrier()

        for acc, S, out in ((accn, NN, dn), (acce, EE, de)):
            base_rows, tail = plan(S)
            row0 = s * base_rows

            def wout(r0, sz, acc=acc, out=out):
                pltpu.sync_copy(acc.at[pl.ds(r0, sz)], rb0.at[pl.ds(0, sz)])
                pltpu.sync_copy(rb0.at[pl.ds(0, sz)], out.at[c, pl.ds(r0, sz)])

            for off in range(0, base_rows, K):
                wout(row0 + off, min(K, base_rows - off))
            if tail:
                @pl.when(s == NS - 1)
                def _():
                    wout(NS * base_rows, tail)

    return k


def _safe_inv(d):
    return jnp.where(d > 0, 1.0 / jnp.where(d > 0, d, 1.0), 0.0)


def _matmul_t(x, W):
    n, f = x.shape
    blk = 1000
    assert n % blk == 0

    def body(x_ref, w_ref, o_ref):
        o_ref[...] = lax.dot_general(
            x_ref[...], w_ref[...], (((1,), (1,)), ((), ())),
            preferred_element_type=jnp.float32,
            precision=lax.Precision.HIGHEST)

    return pl.pallas_call(
        body,
        grid=(n // blk,),
        in_specs=[pl.BlockSpec((blk, f), lambda i: (i, 0)),
                  pl.BlockSpec((f, f), lambda i: (0, 0))],
        out_specs=pl.BlockSpec((blk, f), lambda i: (i, 0)),
        out_shape=jax.ShapeDtypeStruct((n, f), jnp.float32),
    )(x, W)


def _combine_scale(halves, deg_parts):
    _, s, dw = halves.shape
    d = 2 * dw
    dd = deg_parts.shape[2]
    blk = 1000
    assert s % blk == 0

    def body(p_ref, dg_ref, o_ref):
        deg = dg_ref[0, :, 0] + dg_ref[1, :, 0]
        full = jnp.concatenate([p_ref[0], p_ref[1]], axis=1)
        o_ref[...] = full * _safe_inv(deg)[:, None]

    return pl.pallas_call(
        body,
        grid=(s // blk,),
        in_specs=[pl.BlockSpec((2, blk, dw), lambda i: (0, i, 0)),
                  pl.BlockSpec((2, blk, dd), lambda i: (0, i, 0))],
        out_specs=pl.BlockSpec((blk, d), lambda i: (i, 0)),
        out_shape=jax.ShapeDtypeStruct((s, d), jnp.float32),
    )(halves, deg_parts)


def _combine_bias_tanh_stats(halves, deg_parts, b2):
    _, n, dw = halves.shape
    d = 2 * dw
    dd = deg_parts.shape[2]
    blk = 1000
    assert n % blk == 0

    def body(p_ref, dg_ref, b_ref, t_ref, s_ref):
        i = pl.program_id(0)
        deg = dg_ref[0, :, 0] + dg_ref[1, :, 0]
        full = jnp.concatenate([p_ref[0], p_ref[1]], axis=1)
        z = full * _safe_inv(deg)[:, None] + b_ref[...]
        t = jnp.tanh(z)
        t_ref[...] = t
        st = jnp.concatenate(
            [jnp.sum(t, 0, keepdims=True),
             jnp.sum(t * t, 0, keepdims=True),
             jnp.zeros((6, d), jnp.float32)], axis=0)

        @pl.when(i == 0)
        def _():
            s_ref[...] = jnp.zeros_like(s_ref)

        s_ref[...] = s_ref[...] + st

    return pl.pallas_call(
        body,
        grid=(n // blk,),
        in_specs=[pl.BlockSpec((2, blk, dw), lambda i: (0, i, 0)),
                  pl.BlockSpec((2, blk, dd), lambda i: (0, i, 0)),
                  pl.BlockSpec((1, d), lambda i: (0, 0))],
        out_specs=[pl.BlockSpec((blk, d), lambda i: (i, 0)),
                   pl.BlockSpec((8, d), lambda i: (0, 0))],
        out_shape=[jax.ShapeDtypeStruct((n, d), jnp.float32),
                   jax.ShapeDtypeStruct((8, d), jnp.float32)],
    )(halves, deg_parts, b2)


def _batchnorm_apply(t, sums, g2, beta2):
    n, d = t.shape
    blk = 1000
    assert n % blk == 0
    inv_n = 1.0 / n

    def body(t_ref, s_ref, g_ref, be_ref, o_ref):
        m = s_ref[0, :] * inv_n
        v = s_ref[1, :] * inv_n - m * m
        scale = lax.rsqrt(v + EPS) * g_ref[0, :]
        o_ref[...] = (t_ref[...] - m[None, :]) * scale[None, :] + be_ref[...]

    return pl.pallas_call(
        body,
        grid=(n // blk,),
        in_specs=[pl.BlockSpec((blk, d), lambda i: (i, 0)),
                  pl.BlockSpec((8, d), lambda i: (0, 0)),
                  pl.BlockSpec((1, d), lambda i: (0, 0)),
                  pl.BlockSpec((1, d), lambda i: (0, 0))],
        out_specs=pl.BlockSpec((blk, d), lambda i: (i, 0)),
        out_shape=jax.ShapeDtypeStruct((n, d), jnp.float32),
    )(t, sums, g2, beta2)


def kernel(x, edge_index, weight, W0, b0, g0, beta0, W1, b1, g1, beta1):
    n, f = x.shape
    nnz = edge_index.shape[1]
    eh = weight.shape[0]
    fw = f // 2
    NC, NS = _sc_dims()
    NW = NC * NS

    def pad_to(idx, nchunks, val):
        npad = nchunks * K - nnz
        return jnp.concatenate([idx, jnp.full((npad,), val, jnp.int32)])

    cw16 = -(-nnz // (NS * K))
    cw16 += 1 - cw16 % 2
    g_ch = _cwl(cw16) + 1
    s_ch = _cwl(cw16)

    def ext(a, nch, val):
        pad = jnp.full((NS, nch - cw16, K), val, jnp.int32)
        return jnp.concatenate([a, pad], axis=1)

    src16_g = ext(pad_to(edge_index[0], NS * cw16, 0).reshape(NS, cw16, K), g_ch, 0)
    src16_s = ext(pad_to(edge_index[0], NS * cw16, n).reshape(NS, cw16, K), s_ch, n)
    he16_g = ext(pad_to(edge_index[1], NS * cw16, 0).reshape(NS, cw16, K), g_ch, 0)
    he16_s = ext(pad_to(edge_index[1], NS * cw16, eh).reshape(NS, cw16, K), s_ch, eh)
    src_cg = jnp.stack([2 * src16_g, 2 * src16_g + 1])
    he_cg = jnp.stack([2 * he16_g, 2 * he16_g + 1])

    zeros_h = jnp.zeros((K, fw), jnp.float32)

    cw32 = -(-nnz // (NW * K))
    cw32 += 1 - cw32 % 2
    he32 = pad_to(edge_index[1], NW * cw32, eh).reshape(NW, cw32, K)
    src32 = pad_to(edge_index[0], NW * cw32, n).reshape(NW, cw32, K)
    tab_w16 = jnp.zeros((eh + K, 16), jnp.float32).at[:eh, 0].set(weight)
    ones16 = jnp.ones((K, 16), jnp.float32)
    zeros16 = jnp.zeros((K, 16), jnp.float32)

    seg = _seg_sum_rows(n, eh, cw16, f, True)

    dn_parts, de_parts = _degrees(n, eh, cw32)(
        tab_w16, he32, src32, ones16, zeros16)

    def layer(h, W, b, g, beta):
        xl = _matmul_t(h, W)
        pe = seg(xl.reshape(2 * n, fw), src_cg, he16_s, zeros_h)
        out_e = _combine_scale(pe, de_parts)
        pn = seg(out_e.reshape(2 * eh, fw), he_cg, src16_s, zeros_h)
        t, sums = _combine_bias_tanh_stats(pn, dn_parts, b.reshape(1, f))
        return _batchnorm_apply(t, sums, g.reshape(1, f), beta.reshape(1, f))

    h1 = layer(x, W0, b0, g0, beta0)
    h2 = layer(h1, W1, b1, g1, beta1)
    return jnp.stack([h1, h2])


# speedup vs baseline: 1.3538x; 1.2796x over previous
import functools

import jax
import jax.numpy as jnp
from jax import lax
from jax.experimental import pallas as pl
from jax.experimental.pallas import tpu as pltpu
from jax.experimental.pallas import tpu_sc as plsc

EPS = 1e-5
K = 128


def _cwl(cw):
    return -(-(cw - 2) // 4) * 4 + 2


def _sc_dims():
    try:
        info = plsc.get_sparse_core_info()
        return info.num_cores, info.num_subcores
    except ValueError:
        return 2, 16


@functools.lru_cache(maxsize=None)
def _seg_sum_rows(T, S, CW, D, col_split):
    NC, NS = _sc_dims()
    assert S % 8 == 0
    DW = D // 2 if col_split else D
    base_rows = (S // NS) // 8 * 8
    tail = S - NS * base_rows
    mesh = plsc.VectorSubcoreMesh(core_axis_name="c", subcore_axis_name="s",
                                  num_cores=NC, num_subcores=NS)
    gshape = (NC, NS, CW + 3, K) if col_split else (NC * NS, CW + 3, K)
    sshape = (NS, CW, K) if col_split else (NC * NS, CW, K)

    @functools.partial(
        pl.kernel,
        out_type=jax.ShapeDtypeStruct((NC, S, DW), jnp.float32),
        mesh=mesh,
        scratch_types=[
            pltpu.VMEM((_cwl(CW) + 1, K), jnp.int32),
            pltpu.VMEM((_cwl(CW), K), jnp.int32),
            pltpu.VMEM((K, DW), jnp.float32),
            pltpu.VMEM((K, DW), jnp.float32),
            pltpu.VMEM((K, DW), jnp.float32),
            pltpu.VMEM_SHARED((S + K, DW), jnp.float32),
            pltpu.SemaphoreType.DMA,
        ],
        compiler_params=pltpu.CompilerParams(use_tc_tiling_on_sc=False),
    )
    def k(tab, gidx, sidx, zeros, out, gbuf, sbuf, rb0, rb1, zbuf, acc, semg):
        c = lax.axis_index("c")
        s = lax.axis_index("s")

        pltpu.sync_copy(zeros, zbuf)
        row0 = s * base_rows
        for off in range(0, base_rows, K):
            sz = min(K, base_rows - off)
            pltpu.sync_copy(zbuf.at[pl.ds(0, sz)], acc.at[pl.ds(row0 + off, sz)])
        if tail:
            @pl.when(s == NS - 1)
            def _():
                pltpu.sync_copy(zbuf.at[pl.ds(0, tail)],
                                acc.at[pl.ds(NS * base_rows, tail)])
        plsc.subcore_barrier()

        if col_split:
            pltpu.sync_copy(gidx.at[c, s], gbuf)
            pltpu.sync_copy(sidx.at[s], sbuf)
        else:
            w = s * NC + c
            pltpu.sync_copy(gidx.at[w], gbuf)
            pltpu.sync_copy(sidx.at[w], sbuf)

        def start_g(j, buf):
            pltpu.async_copy(tab.at[gbuf.at[j]], buf, semg)

        def wait_g(j, buf):
            pltpu.make_async_copy(tab.at[gbuf.at[j]], buf, semg).wait()

        def scat(j, buf):
            pltpu.sync_copy(buf, acc.at[sbuf.at[j]], add=True)

        start_g(0, rb0)

        def body(i, carry):
            j = 2 * i
            wait_g(j, rb0)
            start_g(j + 1, rb1)
            scat(j, rb0)
            wait_g(j + 1, rb1)
            start_g(j + 2, rb0)
            scat(j + 1, rb1)
            return carry

        lax.fori_loop(0, (CW - 1) // 2, body, 0)
        wait_g(CW - 1, rb0)
        scat(CW - 1, rb0)
        plsc.subcore_barrier()

        def wout(r0, sz):
            pltpu.sync_copy(acc.at[pl.ds(r0, sz)], rb0.at[pl.ds(0, sz)])
            pltpu.sync_copy(rb0.at[pl.ds(0, sz)], out.at[c, pl.ds(r0, sz)])

        for off in range(0, base_rows, K):
            wout(row0 + off, min(K, base_rows - off))
        if tail:
            @pl.when(s == NS - 1)
            def _():
                wout(NS * base_rows, tail)

    return k


@functools.lru_cache(maxsize=None)
def _degrees(NN, EE, CW):
    NC, NS = _sc_dims()
    NW = NC * NS
    DD = 16
    assert CW % 2 == 1

    def plan(S):
        base_rows = (S // NS) // 8 * 8
        return base_rows, S - NS * base_rows

    mesh = plsc.VectorSubcoreMesh(core_axis_name="c", subcore_axis_name="s",
                                  num_cores=NC, num_subcores=NS)

    @functools.partial(
        pl.kernel,
        out_type=(jax.ShapeDtypeStruct((NC, NN, DD), jnp.float32),
                  jax.ShapeDtypeStruct((NC, EE, DD), jnp.float32)),
        mesh=mesh,
        scratch_types=[
            pltpu.VMEM((CW, K), jnp.int32),
            pltpu.VMEM((CW, K), jnp.int32),
            pltpu.VMEM((K, DD), jnp.float32),
            pltpu.VMEM((K, DD), jnp.float32),
            pltpu.VMEM((K, DD), jnp.float32),
            pltpu.VMEM((K, DD), jnp.float32),
            pltpu.VMEM_SHARED((NN + K, DD), jnp.float32),
            pltpu.VMEM_SHARED((EE + K, DD), jnp.float32),
            pltpu.SemaphoreType.DMA,
        ],
        compiler_params=pltpu.CompilerParams(use_tc_tiling_on_sc=False),
    )
    def k(wtab, hidx, sidx, ones, zeros, dn, de, hbuf, sbuf, rb0, rb1, onesb,
          zbuf, accn, acce, sem):
        c = lax.axis_index("c")
        s = lax.axis_index("s")
        w = s * NC + c

        pltpu.sync_copy(zeros, zbuf)
        pltpu.sync_copy(ones, onesb)
        for acc, S in ((accn, NN), (acce, EE)):
            base_rows, tail = plan(S)
            row0 = s * base_rows
            for off in range(0, base_rows, K):
                sz = min(K, base_rows - off)
                pltpu.sync_copy(zbuf.at[pl.ds(0, sz)], acc.at[pl.ds(row0 + off, sz)])
            if tail:
                @pl.when(s == NS - 1)
                def _():
                    pltpu.sync_copy(zbuf.at[pl.ds(0, tail)],
                                    acc.at[pl.ds(NS * base_rows, tail)])
        plsc.subcore_barrier()

        pltpu.sync_copy(hidx.at[w], hbuf)
        pltpu.sync_copy(sidx.at[w], sbuf)

        def start_g(j, buf):
            pltpu.async_copy(wtab.at[hbuf.at[j]], buf, sem)

        def wait_g(j, buf):
            pltpu.make_async_copy(wtab.at[hbuf.at[j]], buf, sem).wait()

        def scat(j, buf):
            pltpu.sync_copy(buf, accn.at[sbuf.at[j]], add=True)
            pltpu.sync_copy(onesb, acce.at[hbuf.at[j]], add=True)

        start_g(0, rb0)

        def body(i, carry):
            j = 2 * i
            wait_g(j, rb0)
            start_g(j + 1, rb1)
            scat(j, rb0)
            wait_g(j + 1, rb1)
            start_g(j + 2, rb0)
            scat(j + 1, rb1)
            return carry

        lax.fori_loop(0, (CW - 1) // 2, body, 0)
        wait_g(CW - 1, rb0)
        scat(CW - 1, rb0)
        plsc.subcore_barrier()

        for acc, S, out in ((accn, NN, dn), (acce, EE, de)):
            base_rows, tail = plan(S)
            row0 = s * base_rows

            def wout(r0, sz, acc=acc, out=out):
                pltpu.sync_copy(acc.at[pl.ds(r0, sz)], rb0.at[pl.ds(0, sz)])
                pltpu.sync_copy(rb0.at[pl.ds(0, sz)], out.at[c, pl.ds(r0, sz)])

            for off in range(0, base_rows, K):
                wout(row0 + off, min(K, base_rows - off))
            if tail:
                @pl.when(s == NS - 1)
                def _():
                    wout(NS * base_rows, tail)

    return k


def _safe_inv(d):
    return jnp.where(d > 0, 1.0 / jnp.where(d > 0, d, 1.0), 0.0)


def _matmul_t(x, W):
    n, f = x.shape
    blk = 1000
    assert n % blk == 0

    def body(x_ref, w_ref, o_ref):
        o_ref[...] = lax.dot_general(
            x_ref[...], w_ref[...], (((1,), (1,)), ((), ())),
            preferred_element_type=jnp.float32,
            precision=lax.Precision.HIGHEST)

    return pl.pallas_call(
        body,
        grid=(n // blk,),
        in_specs=[pl.BlockSpec((blk, f), lambda i: (i, 0)),
                  pl.BlockSpec((f, f), lambda i: (0, 0))],
        out_specs=pl.BlockSpec((blk, f), lambda i: (i, 0)),
        out_shape=jax.ShapeDtypeStruct((n, f), jnp.float32),
    )(x, W)


def _combine_scale(halves, deg_parts):
    _, s, dw = halves.shape
    d = 2 * dw
    dd = deg_parts.shape[2]
    blk = 1000
    assert s % blk == 0

    def body(p_ref, dg_ref, o_ref):
        deg = dg_ref[0, :, 0] + dg_ref[1, :, 0]
        full = jnp.concatenate([p_ref[0], p_ref[1]], axis=1)
        o_ref[...] = full * _safe_inv(deg)[:, None]

    return pl.pallas_call(
        body,
        grid=(s // blk,),
        in_specs=[pl.BlockSpec((2, blk, dw), lambda i: (0, i, 0)),
                  pl.BlockSpec((2, blk, dd), lambda i: (0, i, 0))],
        out_specs=pl.BlockSpec((blk, d), lambda i: (i, 0)),
        out_shape=jax.ShapeDtypeStruct((s, d), jnp.float32),
    )(halves, deg_parts)


def _node_epilogue(halves, deg_parts, b2, g2, beta2):
    _, n, dw = halves.shape
    d = 2 * dw
    dd = deg_parts.shape[2]
    blk = 1000
    assert n % blk == 0
    nb = n // blk
    inv_n = 1.0 / n

    def body(p_ref, dg_ref, b_ref, g_ref, be_ref, o_ref, t_buf, s_buf):
        p = pl.program_id(0)
        i = pl.program_id(1)

        @pl.when(p == 0)
        def _():
            deg = dg_ref[0, :, 0] + dg_ref[1, :, 0]
            full = jnp.concatenate([p_ref[0], p_ref[1]], axis=1)
            t = jnp.tanh(full * _safe_inv(deg)[:, None] + b_ref[...])
            t_buf[pl.ds(i * blk, blk), :] = t
            st = jnp.stack([jnp.sum(t, 0), jnp.sum(t * t, 0)])

            @pl.when(i == 0)
            def _():
                s_buf[...] = jnp.zeros_like(s_buf)

            s_buf[...] = s_buf[...] + st

        @pl.when(p == 1)
        def _():
            m = s_buf[0, :] * inv_n
            v = s_buf[1, :] * inv_n - m * m
            scale = lax.rsqrt(v + EPS) * g_ref[0, :]
            t = t_buf[pl.ds(i * blk, blk), :]
            o_ref[...] = (t - m[None, :]) * scale[None, :] + be_ref[...]

    return pl.pallas_call(
        body,
        grid=(2, nb),
        in_specs=[pl.BlockSpec((2, blk, dw), lambda p, i: (0, i, 0)),
                  pl.BlockSpec((2, blk, dd), lambda p, i: (0, i, 0)),
                  pl.BlockSpec((1, d), lambda p, i: (0, 0)),
                  pl.BlockSpec((1, d), lambda p, i: (0, 0)),
                  pl.BlockSpec((1, d), lambda p, i: (0, 0))],
        out_specs=pl.BlockSpec((blk, d), lambda p, i: (i, 0)),
        out_shape=jax.ShapeDtypeStruct((n, d), jnp.float32),
        scratch_shapes=[pltpu.VMEM((n, d), jnp.float32),
                        pltpu.VMEM((2, d), jnp.float32)],
    )(halves, deg_parts, b2, g2, beta2)


def kernel(x, edge_index, weight, W0, b0, g0, beta0, W1, b1, g1, beta1):
    n, f = x.shape
    nnz = edge_index.shape[1]
    eh = weight.shape[0]
    fw = f // 2
    NC, NS = _sc_dims()
    NW = NC * NS

    def pad_to(idx, nchunks, val):
        npad = nchunks * K - nnz
        return jnp.concatenate([idx, jnp.full((npad,), val, jnp.int32)])

    cw16 = -(-nnz // (NS * K))
    cw16 += 1 - cw16 % 2
    g_ch = _cwl(cw16) + 1
    s_ch = _cwl(cw16)

    def ext(a, nch, val):
        pad = jnp.full((NS, nch - cw16, K), val, jnp.int32)
        return jnp.concatenate([a, pad], axis=1)

    src16_g = ext(pad_to(edge_index[0], NS * cw16, 0).reshape(NS, cw16, K), g_ch, 0)
    src16_s = ext(pad_to(edge_index[0], NS * cw16, n).reshape(NS, cw16, K), s_ch, n)
    he16_g = ext(pad_to(edge_index[1], NS * cw16, 0).reshape(NS, cw16, K), g_ch, 0)
    he16_s = ext(pad_to(edge_index[1], NS * cw16, eh).reshape(NS, cw16, K), s_ch, eh)
    src_cg = jnp.stack([2 * src16_g, 2 * src16_g + 1])
    he_cg = jnp.stack([2 * he16_g, 2 * he16_g + 1])

    zeros_h = jnp.zeros((K, fw), jnp.float32)

    cw32 = -(-nnz // (NW * K))
    cw32 += 1 - cw32 % 2
    he32 = pad_to(edge_index[1], NW * cw32, eh).reshape(NW, cw32, K)
    src32 = pad_to(edge_index[0], NW * cw32, n).reshape(NW, cw32, K)
    tab_w16 = jnp.zeros((eh + K, 16), jnp.float32).at[:eh, 0].set(weight)
    ones16 = jnp.ones((K, 16), jnp.float32)
    zeros16 = jnp.zeros((K, 16), jnp.float32)

    seg = _seg_sum_rows(n, eh, cw16, f, True)

    dn_parts, de_parts = _degrees(n, eh, cw32)(
        tab_w16, he32, src32, ones16, zeros16)

    def layer(h, W, b, g, beta):
        xl = _matmul_t(h, W)
        pe = seg(xl.reshape(2 * n, fw), src_cg, he16_s, zeros_h)
        out_e = _combine_scale(pe, de_parts)
        pn = seg(out_e.reshape(2 * eh, fw), he_cg, src16_s, zeros_h)
        return _node_epilogue(pn, dn_parts, b.reshape(1, f),
                              g.reshape(1, f), beta.reshape(1, f))

    h1 = layer(x, W0, b0, g0, beta0)
    h2 = layer(h1, W1, b1, g1, beta1)
    return jnp.stack([h1, h2])


# speedup vs baseline: 1.3969x; 1.0319x over previous
import functools

import jax
import jax.numpy as jnp
from jax import lax
from jax.experimental import pallas as pl
from jax.experimental.pallas import tpu as pltpu
from jax.experimental.pallas import tpu_sc as plsc

EPS = 1e-5
K = 128


def _cwl(cw):
    return -(-(cw - 2) // 4) * 4 + 2


def _sc_dims():
    try:
        info = plsc.get_sparse_core_info()
        return info.num_cores, info.num_subcores
    except ValueError:
        return 2, 16


@functools.lru_cache(maxsize=None)
def _seg_sum_rows(T, S, CW, D, col_split):
    NC, NS = _sc_dims()
    assert S % 8 == 0
    DW = D // 2 if col_split else D
    base_rows = (S // NS) // 8 * 8
    tail = S - NS * base_rows
    mesh = plsc.VectorSubcoreMesh(core_axis_name="c", subcore_axis_name="s",
                                  num_cores=NC, num_subcores=NS)
    gshape = (NC, NS, CW + 3, K) if col_split else (NC * NS, CW + 3, K)
    sshape = (NS, CW, K) if col_split else (NC * NS, CW, K)

    @functools.partial(
        pl.kernel,
        out_type=jax.ShapeDtypeStruct((NC, S, DW), jnp.float32),
        mesh=mesh,
        scratch_types=[
            pltpu.VMEM((_cwl(CW) + 1, K), jnp.int32),
            pltpu.VMEM((_cwl(CW), K), jnp.int32),
            pltpu.VMEM((K, DW), jnp.float32),
            pltpu.VMEM((K, DW), jnp.float32),
            pltpu.VMEM((K, DW), jnp.float32),
            pltpu.VMEM_SHARED((S + K, DW), jnp.float32),
            pltpu.SemaphoreType.DMA,
        ],
        compiler_params=pltpu.CompilerParams(use_tc_tiling_on_sc=False),
    )
    def k(tab, gidx, sidx, zeros, out, gbuf, sbuf, rb0, rb1, zbuf, acc, semg):
        c = lax.axis_index("c")
        s = lax.axis_index("s")

        pltpu.sync_copy(zeros, zbuf)
        row0 = s * base_rows
        for off in range(0, base_rows, K):
            sz = min(K, base_rows - off)
            pltpu.sync_copy(zbuf.at[pl.ds(0, sz)], acc.at[pl.ds(row0 + off, sz)])
        if tail:
            @pl.when(s == NS - 1)
            def _():
                pltpu.sync_copy(zbuf.at[pl.ds(0, tail)],
                                acc.at[pl.ds(NS * base_rows, tail)])
        plsc.subcore_barrier()

        if col_split:
            pltpu.sync_copy(gidx.at[c, s], gbuf)
            pltpu.sync_copy(sidx.at[s], sbuf)
        else:
            w = s * NC + c
            pltpu.sync_copy(gidx.at[w], gbuf)
            pltpu.sync_copy(sidx.at[w], sbuf)

        def start_g(j, buf):
            pltpu.async_copy(tab.at[gbuf.at[j]], buf, semg)

        def wait_g(j, buf):
            pltpu.make_async_copy(tab.at[gbuf.at[j]], buf, semg).wait()

        def scat(j, buf):
            pltpu.sync_copy(buf, acc.at[sbuf.at[j]], add=True)

        start_g(0, rb0)

        def body(i, carry):
            j = 2 * i
            wait_g(j, rb0)
            start_g(j + 1, rb1)
            scat(j, rb0)
            wait_g(j + 1, rb1)
            start_g(j + 2, rb0)
            scat(j + 1, rb1)
            return carry

        lax.fori_loop(0, (CW - 1) // 2, body, 0)
        wait_g(CW - 1, rb0)
        scat(CW - 1, rb0)
        plsc.subcore_barrier()

        def wout(r0, sz):
            pltpu.sync_copy(acc.at[pl.ds(r0, sz)], rb0.at[pl.ds(0, sz)])
            pltpu.sync_copy(rb0.at[pl.ds(0, sz)], out.at[c, pl.ds(r0, sz)])

        for off in range(0, base_rows, K):
            wout(row0 + off, min(K, base_rows - off))
        if tail:
            @pl.when(s == NS - 1)
            def _():
                wout(NS * base_rows, tail)

    return k


@functools.lru_cache(maxsize=None)
def _seg_sum_scaled(T, S, CW, D, percore_tab):
    NC, NS = _sc_dims()
    assert S % 8 == 0
    DW = D // 2
    base_rows = (S // NS) // 8 * 8
    tail = S - NS * base_rows
    mesh = plsc.VectorSubcoreMesh(core_axis_name="c", subcore_axis_name="s",
                                  num_cores=NC, num_subcores=NS)

    @functools.partial(
        pl.kernel,
        out_type=jax.ShapeDtypeStruct((NC, S, DW), jnp.float32),
        mesh=mesh,
        scratch_types=[
            pltpu.VMEM((_cwl(CW) + 1, K), jnp.int32),
            pltpu.VMEM((_cwl(CW), K), jnp.int32),
            pltpu.VMEM((K, DW), jnp.float32),
            pltpu.VMEM((K, DW), jnp.float32),
            pltpu.VMEM((K, DW), jnp.float32),
            pltpu.VMEM((base_rows + 16, 16), jnp.float32),
            pltpu.VMEM_SHARED((S + K, DW), jnp.float32),
            pltpu.SemaphoreType.DMA,
        ],
        compiler_params=pltpu.CompilerParams(use_tc_tiling_on_sc=False),
    )
    def k(tab, gidx, sidx, inv, zeros, out, gbuf, sbuf, rb0, rb1, zbuf, ibuf,
          acc, semg):
        c = lax.axis_index("c")
        s = lax.axis_index("s")
        row0 = s * base_rows
        last = s == NS - 1

        pltpu.sync_copy(zeros, zbuf)
        for off in range(0, base_rows, K):
            sz = min(K, base_rows - off)
            pltpu.sync_copy(zbuf.at[pl.ds(0, sz)], acc.at[pl.ds(row0 + off, sz)])
        if tail:
            @pl.when(last)
            def _():
                pltpu.sync_copy(zbuf.at[pl.ds(0, tail)],
                                acc.at[pl.ds(NS * base_rows, tail)])

        if percore_tab:
            tsrc = tab.at[c]
            pltpu.sync_copy(gidx.at[s], gbuf)
        else:
            tsrc = tab
            pltpu.sync_copy(gidx.at[c, s], gbuf)
        pltpu.sync_copy(sidx.at[s], sbuf)
        pltpu.sync_copy(inv.at[pl.ds(row0, base_rows)],
                        ibuf.at[pl.ds(0, base_rows)])
        if tail:
            @pl.when(last)
            def _():
                pltpu.sync_copy(inv.at[pl.ds(NS * base_rows, tail)],
                                ibuf.at[pl.ds(base_rows, tail)])
        plsc.subcore_barrier()

        def start_g(j, buf):
            pltpu.async_copy(tsrc.at[gbuf.at[j]], buf, semg)

        def wait_g(j, buf):
            pltpu.make_async_copy(tsrc.at[gbuf.at[j]], buf, semg).wait()

        def scat(j, buf):
            pltpu.sync_copy(buf, acc.at[sbuf.at[j]], add=True)

        start_g(0, rb0)

        def body(i, carry):
            j = 2 * i
            wait_g(j, rb0)
            start_g(j + 1, rb1)
            scat(j, rb0)
            wait_g(j + 1, rb1)
            start_g(j + 2, rb0)
            scat(j + 1, rb1)
            return carry

        lax.fori_loop(0, (CW - 1) // 2, body, 0)
        wait_g(CW - 1, rb0)
        scat(CW - 1, rb0)
        plsc.subcore_barrier()

        def wout(r0, off0, sz):
            pltpu.sync_copy(acc.at[pl.ds(r0, sz)], rb0.at[pl.ds(0, sz)])

            def srow(r, carry):
                sv = ibuf[off0 + r, :]
                for v in range(DW // 16):
                    rb0[r, pl.ds(16 * v, 16)] = rb0[r, pl.ds(16 * v, 16)] * sv
                return carry

            lax.fori_loop(0, sz, srow, 0)
            pltpu.sync_copy(rb0.at[pl.ds(0, sz)], out.at[c, pl.ds(r0, sz)])

        for off in range(0, base_rows, K):
            wout(row0 + off, off, min(K, base_rows - off))
        if tail:
            @pl.when(last)
            def _():
                wout(NS * base_rows, base_rows, tail)

    return k


@functools.lru_cache(maxsize=None)
def _degrees(NN, EE, CW):
    NC, NS = _sc_dims()
    NW = NC * NS
    DD = 16
    assert CW % 2 == 1

    def plan(S):
        base_rows = (S // NS) // 8 * 8
        return base_rows, S - NS * base_rows

    mesh = plsc.VectorSubcoreMesh(core_axis_name="c", subcore_axis_name="s",
                                  num_cores=NC, num_subcores=NS)

    @functools.partial(
        pl.kernel,
        out_type=(jax.ShapeDtypeStruct((NC, NN, DD), jnp.float32),
                  jax.ShapeDtypeStruct((NC, EE, DD), jnp.float32)),
        mesh=mesh,
        scratch_types=[
            pltpu.VMEM((CW, K), jnp.int32),
            pltpu.VMEM((CW, K), jnp.int32),
            pltpu.VMEM((K, DD), jnp.float32),
            pltpu.VMEM((K, DD), jnp.float32),
            pltpu.VMEM((K, DD), jnp.float32),
            pltpu.VMEM((K, DD), jnp.float32),
            pltpu.VMEM_SHARED((NN + K, DD), jnp.float32),
            pltpu.VMEM_SHARED((EE + K, DD), jnp.float32),
            pltpu.SemaphoreType.DMA,
        ],
        compiler_params=pltpu.CompilerParams(use_tc_tiling_on_sc=False),
    )
    def k(wtab, hidx, sidx, ones, zeros, dn, de, hbuf, sbuf, rb0, rb1, onesb,
          zbuf, accn, acce, sem):
        c = lax.axis_index("c")
        s = lax.axis_index("s")
        w = s * NC + c

        pltpu.sync_copy(zeros, zbuf)
        pltpu.sync_copy(ones, onesb)
        for acc, S in ((accn, NN), (acce, EE)):
            base_rows, tail = plan(S)
            row0 = s * base_rows
            for off in range(0, base_rows, K):
                sz = min(K, base_rows - off)
                pltpu.sync_copy(zbuf.at[pl.ds(0, sz)], acc.at[pl.ds(row0 + off, sz)])
            if tail:
                @pl.when(s == NS - 1)
                def _():
                    pltpu.sync_copy(zbuf.at[pl.ds(0, tail)],
                                    acc.at[pl.ds(NS * base_rows, tail)])
        plsc.subcore_barrier()

        pltpu.sync_copy(hidx.at[w], hbuf)
        pltpu.sync_copy(sidx.at[w], sbuf)

        def start_g(j, buf):
            pltpu.async_copy(wtab.at[hbuf.at[j]], buf, sem)

        def wait_g(j, buf):
            pltpu.make_async_copy(wtab.at[hbuf.at[j]], buf, sem).wait()

        def scat(j, buf):
            pltpu.sync_copy(buf, accn.at[sbuf.at[j]], add=True)
            pltpu.sync_copy(onesb, acce.at[hbuf.at[j]], add=True)

        start_g(0, rb0)

        def body(i, carry):
            j = 2 * i
            wait_g(j, rb0)
            start_g(j + 1, rb1)
            scat(j, rb0)
            wait_g(j + 1, rb1)
            start_g(j + 2, rb0)
            scat(j + 1, rb1)
            return carry

        lax.fori_loop(0, (CW - 1) // 2, body, 0)
        wait_g(CW - 1, rb0)
        scat(CW - 1, rb0)
        plsc.subcore_barrier()

        for acc, S, out in ((accn, NN, dn), (acce, EE, de)):
            base_rows, tail = plan(S)
            row0 = s * base_rows

            def wout(r0, sz, acc=acc, out=out):
                pltpu.sync_copy(acc.at[pl.ds(r0, sz)], rb0.at[pl.ds(0, sz)])
                pltpu.sync_copy(rb0.at[pl.ds(0, sz)], out.at[c, pl.ds(r0, sz)])

            for off in range(0, base_rows, K):
                wout(row0 + off, min(K, base_rows - off))
            if tail:
                @pl.when(s == NS - 1)
                def _():
                    wout(NS * base_rows, tail)

    return k


def _safe_inv(d):
    return jnp.where(d > 0, 1.0 / jnp.where(d > 0, d, 1.0), 0.0)


def _matmul_t(x, W):
    n, f = x.shape
    blk = 1000
    assert n % blk == 0

    def body(x_ref, w_ref, o_ref):
        o_ref[...] = lax.dot_general(
            x_ref[...], w_ref[...], (((1,), (1,)), ((), ())),
            preferred_element_type=jnp.float32,
            precision=lax.Precision.HIGHEST)

    return pl.pallas_call(
        body,
        grid=(n // blk,),
        in_specs=[pl.BlockSpec((blk, f), lambda i: (i, 0)),
                  pl.BlockSpec((f, f), lambda i: (0, 0))],
        out_specs=pl.BlockSpec((blk, f), lambda i: (i, 0)),
        out_shape=jax.ShapeDtypeStruct((n, f), jnp.float32),
    )(x, W)


def _inv_degrees(dn_parts, de_parts):
    _, n, dd = dn_parts.shape
    _, eh, _ = de_parts.shape

    def body(dn_ref, de_ref, di_ref, bi_ref):
        di_ref[...] = jnp.broadcast_to(
            _safe_inv(dn_ref[0, :, 0] + dn_ref[1, :, 0])[:, None], (n, dd))
        bi_ref[...] = jnp.broadcast_to(
            _safe_inv(de_ref[0, :, 0] + de_ref[1, :, 0])[:, None], (eh, dd))

    return pl.pallas_call(
        body,
        out_shape=[jax.ShapeDtypeStruct((n, dd), jnp.float32),
                   jax.ShapeDtypeStruct((eh, dd), jnp.float32)],
    )(dn_parts, de_parts)


def _node_epilogue(halves, b2, g2, beta2):
    _, n, dw = halves.shape
    d = 2 * dw
    blk = 1000
    assert n % blk == 0
    nb = n // blk
    inv_n = 1.0 / n

    def body(p_ref, b_ref, g_ref, be_ref, o_ref, t_buf, s_buf):
        p = pl.program_id(0)
        i = pl.program_id(1)

        @pl.when(p == 0)
        def _():
            full = jnp.concatenate([p_ref[0], p_ref[1]], axis=1)
            t = jnp.tanh(full + b_ref[...])
            t_buf[pl.ds(i * blk, blk), :] = t
            st = jnp.stack([jnp.sum(t, 0), jnp.sum(t * t, 0)])

            @pl.when(i == 0)
            def _():
                s_buf[...] = jnp.zeros_like(s_buf)

            s_buf[...] = s_buf[...] + st

        @pl.when(p == 1)
        def _():
            m = s_buf[0, :] * inv_n
            v = s_buf[1, :] * inv_n - m * m
            scale = lax.rsqrt(v + EPS) * g_ref[0, :]
            t = t_buf[pl.ds(i * blk, blk), :]
            o_ref[...] = (t - m[None, :]) * scale[None, :] + be_ref[...]

    return pl.pallas_call(
        body,
        grid=(2, nb),
        in_specs=[pl.BlockSpec((2, blk, dw), lambda p, i: (0, i, 0)),
                  pl.BlockSpec((1, d), lambda p, i: (0, 0)),
                  pl.BlockSpec((1, d), lambda p, i: (0, 0)),
                  pl.BlockSpec((1, d), lambda p, i: (0, 0))],
        out_specs=pl.BlockSpec((blk, d), lambda p, i: (i, 0)),
        out_shape=jax.ShapeDtypeStruct((n, d), jnp.float32),
        scratch_shapes=[pltpu.VMEM((n, d), jnp.float32),
                        pltpu.VMEM((2, d), jnp.float32)],
    )(halves, b2, g2, beta2)


def kernel(x, edge_index, weight, W0, b0, g0, beta0, W1, b1, g1, beta1):
    n, f = x.shape
    nnz = edge_index.shape[1]
    eh = weight.shape[0]
    fw = f // 2
    NC, NS = _sc_dims()
    NW = NC * NS

    def pad_to(idx, nchunks, val):
        npad = nchunks * K - nnz
        return jnp.concatenate([idx, jnp.full((npad,), val, jnp.int32)])

    cw16 = -(-nnz // (NS * K))
    cw16 += 1 - cw16 % 2
    g_ch = _cwl(cw16) + 1
    s_ch = _cwl(cw16)

    def ext(a, nch, val):
        pad = jnp.full((NS, nch - cw16, K), val, jnp.int32)
        return jnp.concatenate([a, pad], axis=1)

    src16_g = ext(pad_to(edge_index[0], NS * cw16, 0).reshape(NS, cw16, K), g_ch, 0)
    src16_s = ext(pad_to(edge_index[0], NS * cw16, n).reshape(NS, cw16, K), s_ch, n)
    he16_g = ext(pad_to(edge_index[1], NS * cw16, 0).reshape(NS, cw16, K), g_ch, 0)
    he16_s = ext(pad_to(edge_index[1], NS * cw16, eh).reshape(NS, cw16, K), s_ch, eh)
    src_cg = jnp.stack([2 * src16_g, 2 * src16_g + 1])

    zeros_h = jnp.zeros((K, fw), jnp.float32)

    cw32 = -(-nnz // (NW * K))
    cw32 += 1 - cw32 % 2
    he32 = pad_to(edge_index[1], NW * cw32, eh).reshape(NW, cw32, K)
    src32 = pad_to(edge_index[0], NW * cw32, n).reshape(NW, cw32, K)
    tab_w16 = jnp.zeros((eh + K, 16), jnp.float32).at[:eh, 0].set(weight)
    ones16 = jnp.ones((K, 16), jnp.float32)
    zeros16 = jnp.zeros((K, 16), jnp.float32)

    seg_e = _seg_sum_scaled(n, eh, cw16, f, False)
    seg_n = _seg_sum_scaled(eh, n, cw16, f, True)

    dn_parts, de_parts = _degrees(n, eh, cw32)(
        tab_w16, he32, src32, ones16, zeros16)
    dinv, binv = _inv_degrees(dn_parts, de_parts)

    def layer(h, W, b, g, beta):
        xl = _matmul_t(h, W)
        oe = seg_e(xl.reshape(2 * n, fw), src_cg, he16_s, binv, zeros_h)
        pn = seg_n(oe, he16_g, src16_s, dinv, zeros_h)
        return _node_epilogue(pn, b.reshape(1, f),
                              g.reshape(1, f), beta.reshape(1, f))

    h1 = layer(x, W0, b0, g0, beta0)
    h2 = layer(h1, W1, b1, g1, beta1)
    return jnp.stack([h1, h2])


# speedup vs baseline: 1.4093x; 1.0089x over previous
import functools

import jax
import jax.numpy as jnp
from jax import lax
from jax.experimental import pallas as pl
from jax.experimental.pallas import tpu as pltpu
from jax.experimental.pallas import tpu_sc as plsc

EPS = 1e-5
K = 128


def _cwl(cw):
    return -(-(cw - 2) // 4) * 4 + 2


def _sc_dims():
    try:
        info = plsc.get_sparse_core_info()
        return info.num_cores, info.num_subcores
    except ValueError:
        return 2, 16


@functools.lru_cache(maxsize=None)
def _seg_sum_rows(T, S, CW, D, col_split):
    NC, NS = _sc_dims()
    assert S % 8 == 0
    DW = D // 2 if col_split else D
    base_rows = (S // NS) // 8 * 8
    tail = S - NS * base_rows
    mesh = plsc.VectorSubcoreMesh(core_axis_name="c", subcore_axis_name="s",
                                  num_cores=NC, num_subcores=NS)
    gshape = (NC, NS, CW + 3, K) if col_split else (NC * NS, CW + 3, K)
    sshape = (NS, CW, K) if col_split else (NC * NS, CW, K)

    @functools.partial(
        pl.kernel,
        out_type=jax.ShapeDtypeStruct((NC, S, DW), jnp.float32),
        mesh=mesh,
        scratch_types=[
            pltpu.VMEM((_cwl(CW) + 1, K), jnp.int32),
            pltpu.VMEM((_cwl(CW), K), jnp.int32),
            pltpu.VMEM((K, DW), jnp.float32),
            pltpu.VMEM((K, DW), jnp.float32),
            pltpu.VMEM((K, DW), jnp.float32),
            pltpu.VMEM_SHARED((S + K, DW), jnp.float32),
            pltpu.SemaphoreType.DMA,
        ],
        compiler_params=pltpu.CompilerParams(use_tc_tiling_on_sc=False),
    )
    def k(tab, gidx, sidx, zeros, out, gbuf, sbuf, rb0, rb1, zbuf, acc, semg):
        c = lax.axis_index("c")
        s = lax.axis_index("s")

        pltpu.sync_copy(zeros, zbuf)
        row0 = s * base_rows
        for off in range(0, base_rows, K):
            sz = min(K, base_rows - off)
            pltpu.sync_copy(zbuf.at[pl.ds(0, sz)], acc.at[pl.ds(row0 + off, sz)])
        if tail:
            @pl.when(s == NS - 1)
            def _():
                pltpu.sync_copy(zbuf.at[pl.ds(0, tail)],
                                acc.at[pl.ds(NS * base_rows, tail)])
        plsc.subcore_barrier()

        if col_split:
            pltpu.sync_copy(gidx.at[c, s], gbuf)
            pltpu.sync_copy(sidx.at[s], sbuf)
        else:
            w = s * NC + c
            pltpu.sync_copy(gidx.at[w], gbuf)
            pltpu.sync_copy(sidx.at[w], sbuf)

        def start_g(j, buf):
            pltpu.async_copy(tab.at[gbuf.at[j]], buf, semg)

        def wait_g(j, buf):
            pltpu.make_async_copy(tab.at[gbuf.at[j]], buf, semg).wait()

        def scat(j, buf):
            pltpu.sync_copy(buf, acc.at[sbuf.at[j]], add=True)

        start_g(0, rb0)

        def body(i, carry):
            j = 2 * i
            wait_g(j, rb0)
            start_g(j + 1, rb1)
            scat(j, rb0)
            wait_g(j + 1, rb1)
            start_g(j + 2, rb0)
            scat(j + 1, rb1)
            return carry

        lax.fori_loop(0, (CW - 1) // 2, body, 0)
        wait_g(CW - 1, rb0)
        scat(CW - 1, rb0)
        plsc.subcore_barrier()

        def wout(r0, sz):
            pltpu.sync_copy(acc.at[pl.ds(r0, sz)], rb0.at[pl.ds(0, sz)])
            pltpu.sync_copy(rb0.at[pl.ds(0, sz)], out.at[c, pl.ds(r0, sz)])

        for off in range(0, base_rows, K):
            wout(row0 + off, min(K, base_rows - off))
        if tail:
            @pl.when(s == NS - 1)
            def _():
                wout(NS * base_rows, tail)

    return k


@functools.lru_cache(maxsize=None)
def _seg_sum_scaled(T, S, CW, D, percore_tab):
    NC, NS = _sc_dims()
    assert S % 8 == 0
    DW = D // 2
    base_rows = (S // NS) // 8 * 8
    tail = S - NS * base_rows
    mesh = plsc.VectorSubcoreMesh(core_axis_name="c", subcore_axis_name="s",
                                  num_cores=NC, num_subcores=NS)

    @functools.partial(
        pl.kernel,
        out_type=jax.ShapeDtypeStruct((NC, S, DW), jnp.float32),
        mesh=mesh,
        scratch_types=[
            pltpu.VMEM((_cwl(CW) + 1, K), jnp.int32),
            pltpu.VMEM((_cwl(CW), K), jnp.int32),
            pltpu.VMEM((K, DW), jnp.float32),
            pltpu.VMEM((K, DW), jnp.float32),
            pltpu.VMEM((K, DW), jnp.float32),
            pltpu.VMEM((base_rows + 16, 16), jnp.float32),
            pltpu.VMEM_SHARED((S + K, DW), jnp.float32),
            pltpu.SemaphoreType.DMA,
        ],
        compiler_params=pltpu.CompilerParams(use_tc_tiling_on_sc=False),
    )
    def k(tab, gidx, sidx, inv, zeros, out, gbuf, sbuf, rb0, rb1, zbuf, ibuf,
          acc, semg):
        c = lax.axis_index("c")
        s = lax.axis_index("s")
        row0 = s * base_rows
        last = s == NS - 1

        pltpu.sync_copy(zeros, zbuf)
        for off in range(0, base_rows, K):
            sz = min(K, base_rows - off)
            pltpu.sync_copy(zbuf.at[pl.ds(0, sz)], acc.at[pl.ds(row0 + off, sz)])
        if tail:
            @pl.when(last)
            def _():
                pltpu.sync_copy(zbuf.at[pl.ds(0, tail)],
                                acc.at[pl.ds(NS * base_rows, tail)])

        if percore_tab:
            tsrc = tab.at[c]
            pltpu.sync_copy(gidx.at[s], gbuf)
        else:
            tsrc = tab
            pltpu.sync_copy(gidx.at[c, s], gbuf)
        pltpu.sync_copy(sidx.at[s], sbuf)
        pltpu.sync_copy(inv.at[pl.ds(row0, base_rows)],
                        ibuf.at[pl.ds(0, base_rows)])
        if tail:
            @pl.when(last)
            def _():
                pltpu.sync_copy(inv.at[pl.ds(NS * base_rows, tail)],
                                ibuf.at[pl.ds(base_rows, tail)])
        plsc.subcore_barrier()

        def start_g(j, buf):
            pltpu.async_copy(tsrc.at[gbuf.at[j]], buf, semg)

        def wait_g(j, buf):
            pltpu.make_async_copy(tsrc.at[gbuf.at[j]], buf, semg).wait()

        def scat(j, buf):
            pltpu.sync_copy(buf, acc.at[sbuf.at[j]], add=True)

        start_g(0, rb0)

        def body(i, carry):
            j = 2 * i
            wait_g(j, rb0)
            start_g(j + 1, rb1)
            scat(j, rb0)
            wait_g(j + 1, rb1)
            start_g(j + 2, rb0)
            scat(j + 1, rb1)
            return carry

        lax.fori_loop(0, (CW - 1) // 2, body, 0)
        wait_g(CW - 1, rb0)
        scat(CW - 1, rb0)
        plsc.subcore_barrier()

        def wout(r0, off0, sz):
            pltpu.sync_copy(acc.at[pl.ds(r0, sz)], rb0.at[pl.ds(0, sz)])

            def srow(r, carry):
                sv = ibuf[off0 + r, :]
                for v in range(DW // 16):
                    rb0[r, pl.ds(16 * v, 16)] = rb0[r, pl.ds(16 * v, 16)] * sv
                return carry

            lax.fori_loop(0, sz, srow, 0)
            pltpu.sync_copy(rb0.at[pl.ds(0, sz)], out.at[c, pl.ds(r0, sz)])

        for off in range(0, base_rows, K):
            wout(row0 + off, off, min(K, base_rows - off))
        if tail:
            @pl.when(last)
            def _():
                wout(NS * base_rows, base_rows, tail)

    return k


@functools.lru_cache(maxsize=None)
def _degrees(NN, EE, CW):
    NC, NS = _sc_dims()
    NW = NC * NS
    DD = 16
    assert CW % 2 == 1

    def plan(S):
        base_rows = (S // NS) // 8 * 8
        return base_rows, S - NS * base_rows

    mesh = plsc.VectorSubcoreMesh(core_axis_name="c", subcore_axis_name="s",
                                  num_cores=NC, num_subcores=NS)

    @functools.partial(
        pl.kernel,
        out_type=(jax.ShapeDtypeStruct((NC, NN, DD), jnp.float32),
                  jax.ShapeDtypeStruct((NC, EE, DD), jnp.float32)),
        mesh=mesh,
        scratch_types=[
            pltpu.VMEM((CW, K), jnp.int32),
            pltpu.VMEM((CW, K), jnp.int32),
            pltpu.VMEM((K, DD), jnp.float32),
            pltpu.VMEM((K, DD), jnp.float32),
            pltpu.VMEM((K, DD), jnp.float32),
            pltpu.VMEM((K, DD), jnp.float32),
            pltpu.VMEM_SHARED((NN + K, DD), jnp.float32),
            pltpu.VMEM_SHARED((EE + K, DD), jnp.float32),
            pltpu.SemaphoreType.DMA,
        ],
        compiler_params=pltpu.CompilerParams(use_tc_tiling_on_sc=False),
    )
    def k(wtab, hidx, sidx, ones, zeros, dn, de, hbuf, sbuf, rb0, rb1, onesb,
          zbuf, accn, acce, sem):
        c = lax.axis_index("c")
        s = lax.axis_index("s")
        w = s * NC + c

        pltpu.sync_copy(zeros, zbuf)
        pltpu.sync_copy(ones, onesb)
        for acc, S in ((accn, NN), (acce, EE)):
            base_rows, tail = plan(S)
            row0 = s * base_rows
            for off in range(0, base_rows, K):
                sz = min(K, base_rows - off)
                pltpu.sync_copy(zbuf.at[pl.ds(0, sz)], acc.at[pl.ds(row0 + off, sz)])
            if tail:
                @pl.when(s == NS - 1)
                def _():
                    pltpu.sync_copy(zbuf.at[pl.ds(0, tail)],
                                    acc.at[pl.ds(NS * base_rows, tail)])
        plsc.subcore_barrier()

        pltpu.sync_copy(hidx.at[w], hbuf)
        pltpu.sync_copy(sidx.at[w], sbuf)

        def start_g(j, buf):
            pltpu.async_copy(wtab.at[hbuf.at[j]], buf, sem)

        def wait_g(j, buf):
            pltpu.make_async_copy(wtab.at[hbuf.at[j]], buf, sem).wait()

        def scat(j, buf):
            pltpu.sync_copy(buf, accn.at[sbuf.at[j]], add=True)
            pltpu.sync_copy(onesb, acce.at[hbuf.at[j]], add=True)

        start_g(0, rb0)

        def body(i, carry):
            j = 2 * i
            wait_g(j, rb0)
            start_g(j + 1, rb1)
            scat(j, rb0)
            wait_g(j + 1, rb1)
            start_g(j + 2, rb0)
            scat(j + 1, rb1)
            return carry

        lax.fori_loop(0, (CW - 1) // 2, body, 0)
        wait_g(CW - 1, rb0)
        scat(CW - 1, rb0)
        plsc.subcore_barrier()

        for acc, S, out in ((accn, NN, dn), (acce, EE, de)):
            base_rows, tail = plan(S)
            row0 = s * base_rows

            def wout(r0, sz, acc=acc, out=out):
                pltpu.sync_copy(acc.at[pl.ds(r0, sz)], rb0.at[pl.ds(0, sz)])
                pltpu.sync_copy(rb0.at[pl.ds(0, sz)], out.at[c, pl.ds(r0, sz)])

            for off in range(0, base_rows, K):
                wout(row0 + off, min(K, base_rows - off))
            if tail:
                @pl.when(s == NS - 1)
                def _():
                    wout(NS * base_rows, tail)

    return k


def _safe_inv(d):
    return jnp.where(d > 0, 1.0 / jnp.where(d > 0, d, 1.0), 0.0)


def _matmul_t(x, W):
    n, f = x.shape
    blk = 1000
    assert n % blk == 0

    def body(x_ref, w_ref, o_ref):
        o_ref[...] = lax.dot_general(
            x_ref[...], w_ref[...], (((1,), (1,)), ((), ())),
            preferred_element_type=jnp.float32,
            precision=lax.Precision.HIGHEST)

    return pl.pallas_call(
        body,
        grid=(n // blk,),
        in_specs=[pl.BlockSpec((blk, f), lambda i: (i, 0)),
                  pl.BlockSpec((f, f), lambda i: (0, 0))],
        out_specs=pl.BlockSpec((blk, f), lambda i: (i, 0)),
        out_shape=jax.ShapeDtypeStruct((n, f), jnp.float32),
    )(x, W)


def _inv_degrees(dn_parts, de_parts):
    _, n, dd = dn_parts.shape
    _, eh, _ = de_parts.shape

    def body(dn_ref, de_ref, di_ref, bi_ref):
        di_ref[...] = jnp.broadcast_to(
            _safe_inv(dn_ref[0, :, 0] + dn_ref[1, :, 0])[:, None], (n, dd))
        bi_ref[...] = jnp.broadcast_to(
            _safe_inv(de_ref[0, :, 0] + de_ref[1, :, 0])[:, None], (eh, dd))

    return pl.pallas_call(
        body,
        out_shape=[jax.ShapeDtypeStruct((n, dd), jnp.float32),
                   jax.ShapeDtypeStruct((eh, dd), jnp.float32)],
    )(dn_parts, de_parts)


def _node_epilogue(halves, b2, g2, beta2, Wn=None):
    _, n, dw = halves.shape
    d = 2 * dw
    blk = 1000
    assert n % blk == 0
    nb = n // blk
    inv_n = 1.0 / n
    with_mm = Wn is not None

    def body(*refs):
        if with_mm:
            p_ref, b_ref, g_ref, be_ref, w_ref, o_ref, xl_ref, t_buf, s_buf = refs
        else:
            p_ref, b_ref, g_ref, be_ref, o_ref, t_buf, s_buf = refs
        p = pl.program_id(0)
        i = pl.program_id(1)

        @pl.when(p == 0)
        def _():
            full = jnp.concatenate([p_ref[0], p_ref[1]], axis=1)
            t = jnp.tanh(full + b_ref[...])
            t_buf[pl.ds(i * blk, blk), :] = t
            st = jnp.stack([jnp.sum(t, 0), jnp.sum(t * t, 0)])

            @pl.when(i == 0)
            def _():
                s_buf[...] = jnp.zeros_like(s_buf)

            s_buf[...] = s_buf[...] + st

        @pl.when(p == 1)
        def _():
            m = s_buf[0, :] * inv_n
            v = s_buf[1, :] * inv_n - m * m
            scale = lax.rsqrt(v + EPS) * g_ref[0, :]
            t = t_buf[pl.ds(i * blk, blk), :]
            h = (t - m[None, :]) * scale[None, :] + be_ref[...]
            o_ref[...] = h
            if with_mm:
                xl_ref[...] = lax.dot_general(
                    h, w_ref[...], (((1,), (1,)), ((), ())),
                    preferred_element_type=jnp.float32,
                    precision=lax.Precision.HIGHEST)

    in_specs = [pl.BlockSpec((2, blk, dw), lambda p, i: (0, i, 0)),
                pl.BlockSpec((1, d), lambda p, i: (0, 0)),
                pl.BlockSpec((1, d), lambda p, i: (0, 0)),
                pl.BlockSpec((1, d), lambda p, i: (0, 0))]
    out_specs = [pl.BlockSpec((blk, d), lambda p, i: (i, 0))]
    out_shape = [jax.ShapeDtypeStruct((n, d), jnp.float32)]
    args = [halves, b2, g2, beta2]
    if with_mm:
        in_specs.append(pl.BlockSpec((d, d), lambda p, i: (0, 0)))
        out_specs.append(pl.BlockSpec((blk, d), lambda p, i: (i, 0)))
        out_shape.append(jax.ShapeDtypeStruct((n, d), jnp.float32))
        args.append(Wn)

    res = pl.pallas_call(
        body,
        grid=(2, nb),
        in_specs=in_specs,
        out_specs=out_specs,
        out_shape=out_shape,
        scratch_shapes=[pltpu.VMEM((n, d), jnp.float32),
                        pltpu.VMEM((2, d), jnp.float32)],
    )(*args)
    return res if with_mm else res[0]


def kernel(x, edge_index, weight, W0, b0, g0, beta0, W1, b1, g1, beta1):
    n, f = x.shape
    nnz = edge_index.shape[1]
    eh = weight.shape[0]
    fw = f // 2
    NC, NS = _sc_dims()
    NW = NC * NS

    def pad_to(idx, nchunks, val):
        npad = nchunks * K - nnz
        return jnp.concatenate([idx, jnp.full((npad,), val, jnp.int32)])

    cw16 = -(-nnz // (NS * K))
    cw16 += 1 - cw16 % 2
    g_ch = _cwl(cw16) + 1
    s_ch = _cwl(cw16)

    def ext(a, nch, val):
        pad = jnp.full((NS, nch - cw16, K), val, jnp.int32)
        return jnp.concatenate([a, pad], axis=1)

    src16_g = ext(pad_to(edge_index[0], NS * cw16, 0).reshape(NS, cw16, K), g_ch, 0)
    src16_s = ext(pad_to(edge_index[0], NS * cw16, n).reshape(NS, cw16, K), s_ch, n)
    he16_g = ext(pad_to(edge_index[1], NS * cw16, 0).reshape(NS, cw16, K), g_ch, 0)
    he16_s = ext(pad_to(edge_index[1], NS * cw16, eh).reshape(NS, cw16, K), s_ch, eh)
    src_cg = jnp.stack([2 * src16_g, 2 * src16_g + 1])

    zeros_h = jnp.zeros((K, fw), jnp.float32)

    cw32 = -(-nnz // (NW * K))
    cw32 += 1 - cw32 % 2
    he32 = pad_to(edge_index[1], NW * cw32, eh).reshape(NW, cw32, K)
    src32 = pad_to(edge_index[0], NW * cw32, n).reshape(NW, cw32, K)
    tab_w16 = jnp.zeros((eh + K, 16), jnp.float32).at[:eh, 0].set(weight)
    ones16 = jnp.ones((K, 16), jnp.float32)
    zeros16 = jnp.zeros((K, 16), jnp.float32)

    seg_e = _seg_sum_scaled(n, eh, cw16, f, False)
    seg_n = _seg_sum_scaled(eh, n, cw16, f, True)

    dn_parts, de_parts = _degrees(n, eh, cw32)(
        tab_w16, he32, src32, ones16, zeros16)
    dinv, binv = _inv_degrees(dn_parts, de_parts)

    def sparse_part(xl):
        oe = seg_e(xl.reshape(2 * n, fw), src_cg, he16_s, binv, zeros_h)
        return seg_n(oe, he16_g, src16_s, dinv, zeros_h)

    xl1 = _matmul_t(x, W0)
    pn1 = sparse_part(xl1)
    h1, xl2 = _node_epilogue(pn1, b0.reshape(1, f), g0.reshape(1, f),
                             beta0.reshape(1, f), W1)
    pn2 = sparse_part(xl2)
    h2 = _node_epilogue(pn2, b1.reshape(1, f), g1.reshape(1, f),
                        beta1.reshape(1, f))
    return jnp.stack([h1, h2])


# speedup vs baseline: 1.4138x; 1.0032x over previous
import functools

import jax
import jax.numpy as jnp
from jax import lax
from jax.experimental import pallas as pl
from jax.experimental.pallas import tpu as pltpu
from jax.experimental.pallas import tpu_sc as plsc

EPS = 1e-5
K = 128


def _cwl(cw):
    return -(-(cw - 2) // 4) * 4 + 2


def _sc_dims():
    try:
        info = plsc.get_sparse_core_info()
        return info.num_cores, info.num_subcores
    except ValueError:
        return 2, 16


@functools.lru_cache(maxsize=None)
def _seg_sum_rows(T, S, CW, D, col_split):
    NC, NS = _sc_dims()
    assert S % 8 == 0
    DW = D // 2 if col_split else D
    base_rows = (S // NS) // 8 * 8
    tail = S - NS * base_rows
    mesh = plsc.VectorSubcoreMesh(core_axis_name="c", subcore_axis_name="s",
                                  num_cores=NC, num_subcores=NS)
    gshape = (NC, NS, CW + 3, K) if col_split else (NC * NS, CW + 3, K)
    sshape = (NS, CW, K) if col_split else (NC * NS, CW, K)

    @functools.partial(
        pl.kernel,
        out_type=jax.ShapeDtypeStruct((NC, S, DW), jnp.float32),
        mesh=mesh,
        scratch_types=[
            pltpu.VMEM((_cwl(CW) + 1, K), jnp.int32),
            pltpu.VMEM((_cwl(CW), K), jnp.int32),
            pltpu.VMEM((K, DW), jnp.float32),
            pltpu.VMEM((K, DW), jnp.float32),
            pltpu.VMEM((K, DW), jnp.float32),
            pltpu.VMEM_SHARED((S + K, DW), jnp.float32),
            pltpu.SemaphoreType.DMA,
        ],
        compiler_params=pltpu.CompilerParams(use_tc_tiling_on_sc=False),
    )
    def k(tab, gidx, sidx, zeros, out, gbuf, sbuf, rb0, rb1, zbuf, acc, semg):
        c = lax.axis_index("c")
        s = lax.axis_index("s")

        pltpu.sync_copy(zeros, zbuf)
        row0 = s * base_rows
        for off in range(0, base_rows, K):
            sz = min(K, base_rows - off)
            pltpu.sync_copy(zbuf.at[pl.ds(0, sz)], acc.at[pl.ds(row0 + off, sz)])
        if tail:
            @pl.when(s == NS - 1)
            def _():
                pltpu.sync_copy(zbuf.at[pl.ds(0, tail)],
                                acc.at[pl.ds(NS * base_rows, tail)])
        plsc.subcore_barrier()

        if col_split:
            pltpu.sync_copy(gidx.at[c, s], gbuf)
            pltpu.sync_copy(sidx.at[s], sbuf)
        else:
            w = s * NC + c
            pltpu.sync_copy(gidx.at[w], gbuf)
            pltpu.sync_copy(sidx.at[w], sbuf)

        def start_g(j, buf):
            pltpu.async_copy(tab.at[gbuf.at[j]], buf, semg)

        def wait_g(j, buf):
            pltpu.make_async_copy(tab.at[gbuf.at[j]], buf, semg).wait()

        def scat(j, buf):
            pltpu.sync_copy(buf, acc.at[sbuf.at[j]], add=True)

        start_g(0, rb0)

        def body(i, carry):
            j = 2 * i
            wait_g(j, rb0)
            start_g(j + 1, rb1)
            scat(j, rb0)
            wait_g(j + 1, rb1)
            start_g(j + 2, rb0)
            scat(j + 1, rb1)
            return carry

        lax.fori_loop(0, (CW - 1) // 2, body, 0)
        wait_g(CW - 1, rb0)
        scat(CW - 1, rb0)
        plsc.subcore_barrier()

        def wout(r0, sz):
            pltpu.sync_copy(acc.at[pl.ds(r0, sz)], rb0.at[pl.ds(0, sz)])
            pltpu.sync_copy(rb0.at[pl.ds(0, sz)], out.at[c, pl.ds(r0, sz)])

        for off in range(0, base_rows, K):
            wout(row0 + off, min(K, base_rows - off))
        if tail:
            @pl.when(s == NS - 1)
            def _():
                wout(NS * base_rows, tail)

    return k


@functools.lru_cache(maxsize=None)
def _seg_sum_scaled(T, S, CW, D, percore_tab):
    NC, NS = _sc_dims()
    assert S % 8 == 0
    DW = D // 2
    base_rows = (S // NS) // 8 * 8
    tail = S - NS * base_rows
    mesh = plsc.VectorSubcoreMesh(core_axis_name="c", subcore_axis_name="s",
                                  num_cores=NC, num_subcores=NS)

    @functools.partial(
        pl.kernel,
        out_type=jax.ShapeDtypeStruct((NC, S, DW), jnp.float32),
        mesh=mesh,
        scratch_types=[
            pltpu.VMEM((_cwl(CW) + 1, K), jnp.int32),
            pltpu.VMEM((_cwl(CW), K), jnp.int32),
            pltpu.VMEM((K, DW), jnp.float32),
            pltpu.VMEM((K, DW), jnp.float32),
            pltpu.VMEM((K, DW), jnp.float32),
            pltpu.VMEM((base_rows + 16, 16), jnp.float32),
            pltpu.VMEM_SHARED((S + K, DW), jnp.float32),
            pltpu.SemaphoreType.DMA,
        ],
        compiler_params=pltpu.CompilerParams(use_tc_tiling_on_sc=False),
    )
    def k(tab, gidx, sidx, inv, zeros, out, gbuf, sbuf, rb0, rb1, zbuf, ibuf,
          acc, semg):
        c = lax.axis_index("c")
        s = lax.axis_index("s")
        row0 = s * base_rows
        last = s == NS - 1

        pltpu.sync_copy(zeros, zbuf)
        for off in range(0, base_rows, K):
            sz = min(K, base_rows - off)
            pltpu.sync_copy(zbuf.at[pl.ds(0, sz)], acc.at[pl.ds(row0 + off, sz)])
        if tail:
            @pl.when(last)
            def _():
                pltpu.sync_copy(zbuf.at[pl.ds(0, tail)],
                                acc.at[pl.ds(NS * base_rows, tail)])

        if percore_tab:
            tsrc = tab.at[c]
            pltpu.sync_copy(gidx.at[s], gbuf)
        else:
            tsrc = tab
            pltpu.sync_copy(gidx.at[c, s], gbuf)
        pltpu.sync_copy(sidx.at[s], sbuf)
        pltpu.sync_copy(inv.at[pl.ds(row0, base_rows)],
                        ibuf.at[pl.ds(0, base_rows)])
        if tail:
            @pl.when(last)
            def _():
                pltpu.sync_copy(inv.at[pl.ds(NS * base_rows, tail)],
                                ibuf.at[pl.ds(base_rows, tail)])
        plsc.subcore_barrier()

        def start_g(j, buf):
            pltpu.async_copy(tsrc.at[gbuf.at[j]], buf, semg)

        def wait_g(j, buf):
            pltpu.make_async_copy(tsrc.at[gbuf.at[j]], buf, semg).wait()

        def scat(j, buf):
            pltpu.sync_copy(buf, acc.at[sbuf.at[j]], add=True)

        start_g(0, rb0)

        def body(i, carry):
            j = 2 * i
            wait_g(j, rb0)
            start_g(j + 1, rb1)
            scat(j, rb0)
            wait_g(j + 1, rb1)
            start_g(j + 2, rb0)
            scat(j + 1, rb1)
            return carry

        lax.fori_loop(0, (CW - 1) // 2, body, 0)
        wait_g(CW - 1, rb0)
        scat(CW - 1, rb0)
        plsc.subcore_barrier()

        chunks = [(row0 + off, off, min(K, base_rows - off))
                  for off in range(0, base_rows, K)]

        def stage_in(i, buf):
            r0, _, sz = chunks[i]
            return pltpu.async_copy(acc.at[pl.ds(r0, sz)], buf.at[pl.ds(0, sz)],
                                    semg)

        def wait_in(i, buf):
            r0, _, sz = chunks[i]
            pltpu.make_async_copy(acc.at[pl.ds(r0, sz)], buf.at[pl.ds(0, sz)],
                                  semg).wait()

        def scale_and_out(i, buf):
            r0, off0, sz = chunks[i]

            def srow(r, carry):
                sv = ibuf[off0 + r, :]
                for v in range(DW // 16):
                    buf[r, pl.ds(16 * v, 16)] = buf[r, pl.ds(16 * v, 16)] * sv
                return carry

            lax.fori_loop(0, sz, srow, 0)
            pltpu.sync_copy(buf.at[pl.ds(0, sz)], out.at[c, pl.ds(r0, sz)])

        wbufs = (rb0, rb1)
        stage_in(0, wbufs[0])
        for i in range(len(chunks)):
            wait_in(i, wbufs[i % 2])
            if i + 1 < len(chunks):
                stage_in(i + 1, wbufs[(i + 1) % 2])
            scale_and_out(i, wbufs[i % 2])
        if tail:
            @pl.when(last)
            def _():
                r0, off0, sz = NS * base_rows, base_rows, tail
                pltpu.sync_copy(acc.at[pl.ds(r0, sz)], rb0.at[pl.ds(0, sz)])

                def srow(r, carry):
                    sv = ibuf[off0 + r, :]
                    for v in range(DW // 16):
                        rb0[r, pl.ds(16 * v, 16)] = rb0[r, pl.ds(16 * v, 16)] * sv
                    return carry

                lax.fori_loop(0, sz, srow, 0)
                pltpu.sync_copy(rb0.at[pl.ds(0, sz)], out.at[c, pl.ds(r0, sz)])

    return k


@functools.lru_cache(maxsize=None)
def _degrees(NN, EE, CW):
    NC, NS = _sc_dims()
    NW = NC * NS
    DD = 16
    assert CW % 2 == 1

    def plan(S):
        base_rows = (S // NS) // 8 * 8
        return base_rows, S - NS * base_rows

    mesh = plsc.VectorSubcoreMesh(core_axis_name="c", subcore_axis_name="s",
                                  num_cores=NC, num_subcores=NS)

    @functools.partial(
        pl.kernel,
        out_type=(jax.ShapeDtypeStruct((NC, NN, DD), jnp.float32),
                  jax.ShapeDtypeStruct((NC, EE, DD), jnp.float32)),
        mesh=mesh,
        scratch_types=[
            pltpu.VMEM((CW, K), jnp.int32),
            pltpu.VMEM((CW, K), jnp.int32),
            pltpu.VMEM((K, DD), jnp.float32),
            pltpu.VMEM((K, DD), jnp.float32),
            pltpu.VMEM((K, DD), jnp.float32),
            pltpu.VMEM((K, DD), jnp.float32),
            pltpu.VMEM_SHARED((NN + K, DD), jnp.float32),
            pltpu.VMEM_SHARED((EE + K, DD), jnp.float32),
            pltpu.SemaphoreType.DMA,
        ],
        compiler_params=pltpu.CompilerParams(use_tc_tiling_on_sc=False),
    )
    def k(wtab, hidx, sidx, ones, zeros, dn, de, hbuf, sbuf, rb0, rb1, onesb,
          zbuf, accn, acce, sem):
        c = lax.axis_index("c")
        s = lax.axis_index("s")
        w = s * NC + c

        pltpu.sync_copy(zeros, zbuf)
        pltpu.sync_copy(ones, onesb)
        for acc, S in ((accn, NN), (acce, EE)):
            base_rows, tail = plan(S)
            row0 = s * base_rows
            for off in range(0, base_rows, K):
                sz = min(K, base_rows - off)
                pltpu.sync_copy(zbuf.at[pl.ds(0, sz)], acc.at[pl.ds(row0 + off, sz)])
            if tail:
                @pl.when(s == NS - 1)
                def _():
                    pltpu.sync_copy(zbuf.at[pl.ds(0, tail)],
                                    acc.at[pl.ds(NS * base_rows, tail)])
        plsc.subcore_barrier()

        pltpu.sync_copy(hidx.at[w], hbuf)
        pltpu.sync_copy(sidx.at[w], sbuf)

        def start_g(j, buf):
            pltpu.async_copy(wtab.at[hbuf.at[j]], buf, sem)

        def wait_g(j, buf):
            pltpu.make_async_copy(wtab.at[hbuf.at[j]], buf, sem).wait()

        def scat(j, buf):
            pltpu.sync_copy(buf, accn.at[sbuf.at[j]], add=True)
            pltpu.sync_copy(onesb, acce.at[hbuf.at[j]], add=True)

        start_g(0, rb0)

        def body(i, carry):
            j = 2 * i
            wait_g(j, rb0)
            start_g(j + 1, rb1)
            scat(j, rb0)
            wait_g(j + 1, rb1)
            start_g(j + 2, rb0)
            scat(j + 1, rb1)
            return carry

        lax.fori_loop(0, (CW - 1) // 2, body, 0)
        wait_g(CW - 1, rb0)
        scat(CW - 1, rb0)
        plsc.subcore_barrier()

        for acc, S, out in ((accn, NN, dn), (acce, EE, de)):
            base_rows, tail = plan(S)
            row0 = s * base_rows

            def wout(r0, sz, acc=acc, out=out):
                pltpu.sync_copy(acc.at[pl.ds(r0, sz)], rb0.at[pl.ds(0, sz)])
                pltpu.sync_copy(rb0.at[pl.ds(0, sz)], out.at[c, pl.ds(r0, sz)])

            for off in range(0, base_rows, K):
                wout(row0 + off, min(K, base_rows - off))
            if tail:
                @pl.when(s == NS - 1)
                def _():
                    wout(NS * base_rows, tail)

    return k


def _safe_inv(d):
    return jnp.where(d > 0, 1.0 / jnp.where(d > 0, d, 1.0), 0.0)


def _matmul_t(x, W):
    n, f = x.shape
    blk = 1000
    assert n % blk == 0

    def body(x_ref, w_ref, o_ref):
        o_ref[...] = lax.dot_general(
            x_ref[...], w_ref[...], (((1,), (1,)), ((), ())),
            preferred_element_type=jnp.float32,
            precision=lax.Precision.HIGHEST)

    return pl.pallas_call(
        body,
        grid=(n // blk,),
        in_specs=[pl.BlockSpec((blk, f), lambda i: (i, 0)),
                  pl.BlockSpec((f, f), lambda i: (0, 0))],
        out_specs=pl.BlockSpec((blk, f), lambda i: (i, 0)),
        out_shape=jax.ShapeDtypeStruct((n, f), jnp.float32),
    )(x, W)


def _inv_degrees(dn_parts, de_parts):
    _, n, dd = dn_parts.shape
    _, eh, _ = de_parts.shape

    def body(dn_ref, de_ref, di_ref, bi_ref):
        di_ref[...] = jnp.broadcast_to(
            _safe_inv(dn_ref[0, :, 0] + dn_ref[1, :, 0])[:, None], (n, dd))
        bi_ref[...] = jnp.broadcast_to(
            _safe_inv(de_ref[0, :, 0] + de_ref[1, :, 0])[:, None], (eh, dd))

    return pl.pallas_call(
        body,
        out_shape=[jax.ShapeDtypeStruct((n, dd), jnp.float32),
                   jax.ShapeDtypeStruct((eh, dd), jnp.float32)],
    )(dn_parts, de_parts)


def _node_epilogue(halves, b2, g2, beta2, Wn=None):
    _, n, dw = halves.shape
    d = 2 * dw
    blk = 1000
    assert n % blk == 0
    nb = n // blk
    inv_n = 1.0 / n
    with_mm = Wn is not None

    def body(*refs):
        if with_mm:
            p_ref, b_ref, g_ref, be_ref, w_ref, o_ref, xl_ref, t_buf, s_buf = refs
        else:
            p_ref, b_ref, g_ref, be_ref, o_ref, t_buf, s_buf = refs
        p = pl.program_id(0)
        i = pl.program_id(1)

        @pl.when(p == 0)
        def _():
            full = jnp.concatenate([p_ref[0], p_ref[1]], axis=1)
            t = jnp.tanh(full + b_ref[...])
            t_buf[pl.ds(i * blk, blk), :] = t
            st = jnp.stack([jnp.sum(t, 0), jnp.sum(t * t, 0)])

            @pl.when(i == 0)
            def _():
                s_buf[...] = jnp.zeros_like(s_buf)

            s_buf[...] = s_buf[...] + st

        @pl.when(p == 1)
        def _():
            m = s_buf[0, :] * inv_n
            v = s_buf[1, :] * inv_n - m * m
            scale = lax.rsqrt(v + EPS) * g_ref[0, :]
            t = t_buf[pl.ds(i * blk, blk), :]
            h = (t - m[None, :]) * scale[None, :] + be_ref[...]
            o_ref[...] = h
            if with_mm:
                xl_ref[...] = lax.dot_general(
                    h, w_ref[...], (((1,), (1,)), ((), ())),
                    preferred_element_type=jnp.float32,
                    precision=lax.Precision.HIGHEST)

    in_specs = [pl.BlockSpec((2, blk, dw), lambda p, i: (0, i, 0)),
                pl.BlockSpec((1, d), lambda p, i: (0, 0)),
                pl.BlockSpec((1, d), lambda p, i: (0, 0)),
                pl.BlockSpec((1, d), lambda p, i: (0, 0))]
    out_specs = [pl.BlockSpec((blk, d), lambda p, i: (i, 0))]
    out_shape = [jax.ShapeDtypeStruct((n, d), jnp.float32)]
    args = [halves, b2, g2, beta2]
    if with_mm:
        in_specs.append(pl.BlockSpec((d, d), lambda p, i: (0, 0)))
        out_specs.append(pl.BlockSpec((blk, d), lambda p, i: (i, 0)))
        out_shape.append(jax.ShapeDtypeStruct((n, d), jnp.float32))
        args.append(Wn)

    res = pl.pallas_call(
        body,
        grid=(2, nb),
        in_specs=in_specs,
        out_specs=out_specs,
        out_shape=out_shape,
        scratch_shapes=[pltpu.VMEM((n, d), jnp.float32),
                        pltpu.VMEM((2, d), jnp.float32)],
    )(*args)
    return res if with_mm else res[0]


def kernel(x, edge_index, weight, W0, b0, g0, beta0, W1, b1, g1, beta1):
    n, f = x.shape
    nnz = edge_index.shape[1]
    eh = weight.shape[0]
    fw = f // 2
    NC, NS = _sc_dims()
    NW = NC * NS

    def pad_to(idx, nchunks, val):
        npad = nchunks * K - nnz
        return jnp.concatenate([idx, jnp.full((npad,), val, jnp.int32)])

    cw16 = -(-nnz // (NS * K))
    cw16 += 1 - cw16 % 2
    g_ch = _cwl(cw16) + 1
    s_ch = _cwl(cw16)

    def ext(a, nch, val):
        pad = jnp.full((NS, nch - cw16, K), val, jnp.int32)
        return jnp.concatenate([a, pad], axis=1)

    src16_g = ext(pad_to(edge_index[0], NS * cw16, 0).reshape(NS, cw16, K), g_ch, 0)
    src16_s = ext(pad_to(edge_index[0], NS * cw16, n).reshape(NS, cw16, K), s_ch, n)
    he16_g = ext(pad_to(edge_index[1], NS * cw16, 0).reshape(NS, cw16, K), g_ch, 0)
    he16_s = ext(pad_to(edge_index[1], NS * cw16, eh).reshape(NS, cw16, K), s_ch, eh)
    src_cg = jnp.stack([2 * src16_g, 2 * src16_g + 1])

    zeros_h = jnp.zeros((K, fw), jnp.float32)

    cw32 = -(-nnz // (NW * K))
    cw32 += 1 - cw32 % 2
    he32 = pad_to(edge_index[1], NW * cw32, eh).reshape(NW, cw32, K)
    src32 = pad_to(edge_index[0], NW * cw32, n).reshape(NW, cw32, K)
    tab_w16 = jnp.zeros((eh + K, 16), jnp.float32).at[:eh, 0].set(weight)
    ones16 = jnp.ones((K, 16), jnp.float32)
    zeros16 = jnp.zeros((K, 16), jnp.float32)

    seg_e = _seg_sum_scaled(n, eh, cw16, f, False)
    seg_n = _seg_sum_scaled(eh, n, cw16, f, True)

    dn_parts, de_parts = _degrees(n, eh, cw32)(
        tab_w16, he32, src32, ones16, zeros16)
    dinv, binv = _inv_degrees(dn_parts, de_parts)

    def sparse_part(xl):
        oe = seg_e(xl.reshape(2 * n, fw), src_cg, he16_s, binv, zeros_h)
        return seg_n(oe, he16_g, src16_s, dinv, zeros_h)

    xl1 = _matmul_t(x, W0)
    pn1 = sparse_part(xl1)
    h1, xl2 = _node_epilogue(pn1, b0.reshape(1, f), g0.reshape(1, f),
                             beta0.reshape(1, f), W1)
    pn2 = sparse_part(xl2)
    h2 = _node_epilogue(pn2, b1.reshape(1, f), g1.reshape(1, f),
                        beta1.reshape(1, f))
    return jnp.stack([h1, h2])
